# Initial kernel scaffold; baseline (speedup 1.0000x reference)
import jax
import jax.numpy as jnp
from jax.experimental import pallas as pl


def kernel(x, e, Wn1, bn1, Wn2, bn2, We1, be1, We2, be2, W_gnn, b_gnn, ln_ng, ln_nb, ln_eg, ln_eb, Wp1, bp1, Wp2, bp2, edge_index):
    raise NotImplementedError("write your pallas kernel here")


# speedup vs baseline: 1.9708x; 1.9708x over previous
import jax
import jax.numpy as jnp
from jax import lax
from jax.experimental import pallas as pl
from jax.experimental.pallas import tpu as pltpu
from jax.experimental.pallas import tpu_sc as plsc

N = 10000
E = 320000
H = 64
W = 2 * H
EB = 2000
EBP = 2048
NBLK = E // EB
EP = EBP * NBLK
NW = 32
IB = 1024
NCH = EP // (NW * IB)
CH = 512
NH = N // 2
NACC = NH + 8
NPT = 312
ZR = 104

_MESH = dict(core_axis_name="c", subcore_axis_name="s", num_cores=2,
             num_subcores=16)


def _sc_gather_body(tab_hbm, idx_hbm, out_hbm, idxv, buf, sem):
    c = lax.axis_index("c")
    s = lax.axis_index("s")
    wid = s * 2 + c

    def chunk(i, carry):
        b0 = wid * NCH + i
        pltpu.sync_copy(idx_hbm.at[b0], idxv)
        for r in range(IB // CH):
            e0 = b0 * IB + r * CH
            cps = [pltpu.async_copy(tab_hbm.at[idxv.at[r * 4 + j]],
                                    buf.at[pl.ds(j * 128, 128)], sem)
                   for j in range(CH // 128)]
            for cp in cps:
                cp.wait()
            pltpu.sync_copy(buf, out_hbm.at[pl.ds(e0, CH)])
        return carry
    lax.fori_loop(0, NCH, chunk, 0)


def _sc_gather(table, idx3d):
    f = pl.kernel(
        _sc_gather_body,
        out_type=jax.ShapeDtypeStruct((EP, W), jnp.float32),
        mesh=plsc.VectorSubcoreMesh(**_MESH),
        scratch_types=[
            pltpu.VMEM((IB // 128, 128), jnp.int32),
            pltpu.VMEM((CH, W), jnp.float32),
            pltpu.SemaphoreType.DMA,
        ],
    )
    return f(table, idx3d)


def _sc_scatter_body(ms_hbm, idx_hbm, z_hbm, acc_out,
                     idxv, sbuf, acc_sh, sem):
    c = lax.axis_index("c")
    s = lax.axis_index("s")

    r0 = s * NPT
    for k in range(3):
        pltpu.sync_copy(z_hbm, acc_sh.at[pl.ds(r0 + k * ZR, ZR)])

    @pl.when(s == 15)
    def _():
        pltpu.sync_copy(z_hbm.at[pl.ds(0, 8)], acc_sh.at[pl.ds(16 * NPT, 8)])
    plsc.subcore_barrier()

    lo = c * NH

    def chunk(i, carry):
        b0 = s * (2 * NCH) + i
        pltpu.sync_copy(idx_hbm.at[b0], idxv)
        for j in range(IB // 128):
            for q in range(8):
                v = idxv[j, pl.ds(q * 16, 16)] - lo
                ok = (v >= 0) & (v < NH)
                idxv[j, pl.ds(q * 16, 16)] = jnp.where(ok, v, NH)
        for r in range(IB // CH):
            e0 = b0 * IB + r * CH
            pltpu.sync_copy(ms_hbm.at[pl.ds(e0, CH)], sbuf)
            for j in range(CH // 128):
                pltpu.sync_copy(sbuf.at[pl.ds(j * 128, 128)],
                                acc_sh.at[idxv.at[r * 4 + j]], add=True)
        return carry
    lax.fori_loop(0, 2 * NCH, chunk, 0)

    plsc.subcore_barrier()
    pltpu.sync_copy(acc_sh.at[pl.ds(r0, NPT)],
                    acc_out.at[pl.ds(lo + r0, NPT)])

    @pl.when(s == 15)
    def _():
        pltpu.sync_copy(acc_sh.at[pl.ds(16 * NPT, 8)],
                        acc_out.at[pl.ds(lo + 16 * NPT, 8)])


def _sc_scatter(ms_p, idx3d, zrows):
    f = pl.kernel(
        _sc_scatter_body,
        out_type=jax.ShapeDtypeStruct((N, W), jnp.float32),
        mesh=plsc.VectorSubcoreMesh(**_MESH),
        scratch_types=[
            pltpu.VMEM((IB // 128, 128), jnp.int32),
            pltpu.VMEM((CH, W), jnp.float32),
            pltpu.VMEM_SHARED((NACC, W), jnp.float32),
            pltpu.SemaphoreType.DMA,
        ],
    )
    return f(ms_p, idx3d, zrows)


def _ln(v, g, b):
    mu = jnp.mean(v, axis=-1, keepdims=True)
    var = jnp.mean((v - mu) * (v - mu), axis=-1, keepdims=True)
    return g * (v - mu) * lax.rsqrt(var + 1e-5) + b


def _tc_enc_body(x_ref, w1_ref, b1_ref, w2_ref, b2_ref, o_ref):
    hh = jnp.maximum(
        jnp.dot(x_ref[:], w1_ref[:], preferred_element_type=jnp.float32)
        + b1_ref[:], 0.0)
    o_ref[:] = jnp.dot(hh, w2_ref[:], preferred_element_type=jnp.float32) + b2_ref[:]


def _node_enc(x, w1, b1, w2, b2):
    return pl.pallas_call(
        _tc_enc_body,
        out_shape=jax.ShapeDtypeStruct((N, H), jnp.float32),
    )(x, w1, b1, w2, b2)


def _edge_enc(e, w1, b1, w2, b2):
    d_edge = e.shape[1]
    return pl.pallas_call(
        _tc_enc_body,
        grid=(NBLK,),
        in_specs=[
            pl.BlockSpec((EB, d_edge), lambda i: (i, 0)),
            pl.BlockSpec((d_edge, H), lambda i: (0, 0)),
            pl.BlockSpec((1, H), lambda i: (0, 0)),
            pl.BlockSpec((H, H), lambda i: (0, 0)),
            pl.BlockSpec((1, H), lambda i: (0, 0)),
        ],
        out_specs=pl.BlockSpec((EB, H), lambda i: (i, 0)),
        out_shape=jax.ShapeDtypeStruct((E, H), jnp.float32),
    )(e, w1, b1, w2, b2)


def _tc_node_mats_body(h_ref, wa, wb, wd, we, ba, bb, bd, be,
                       ax_o, t1_o, t2_o):
    h = h_ref[:]
    ax_o[:] = jnp.dot(h, wa[:], preferred_element_type=jnp.float32) + ba[:]
    bx = jnp.dot(h, wb[:], preferred_element_type=jnp.float32) + bb[:]
    dx = jnp.dot(h, wd[:], preferred_element_type=jnp.float32) + bd[:]
    ex = jnp.dot(h, we[:], preferred_element_type=jnp.float32) + be[:]
    t1_o[:] = jnp.concatenate([dx, bx], axis=1)
    t2_o[:] = jnp.concatenate([ex, jnp.zeros((N, H), jnp.float32)], axis=1)


def _node_mats(h, wa, wb, wd, we, ba, bb, bd, be):
    return pl.pallas_call(
        _tc_node_mats_body,
        out_shape=(jax.ShapeDtypeStruct((N, H), jnp.float32),
                   jax.ShapeDtypeStruct((N, W), jnp.float32),
                   jax.ShapeDtypeStruct((N, W), jnp.float32)),
    )(h, wa, wb, wd, we, ba, bb, bd, be)


def _tc_edge_update_body(g1_ref, g2_ref, ee_ref, c_ref, bc_ref, eg_ref,
                         eb_ref, eeo_ref, ms_ref):
    ee = ee_ref[:]
    ce = jnp.dot(ee, c_ref[:], preferred_element_type=jnp.float32) + bc_ref[:]
    g1 = g1_ref[0:EB, :]
    epre = g1[:, 0:H] + g2_ref[0:EB, 0:H] + ce
    eeo_ref[:] = ee + jnp.maximum(_ln(epre, eg_ref[:], eb_ref[:]), 0.0)
    sig = jax.nn.sigmoid(epre)
    msg = sig * g1[:, H:W]
    ms_ref[0:EB, :] = jnp.concatenate([msg, sig], axis=1)
    ms_ref[EB:EBP, :] = jnp.zeros((EBP - EB, W), jnp.float32)


def _edge_update(g1_p, g2_p, ee, wc, bc, eg, eb):
    return pl.pallas_call(
        _tc_edge_update_body,
        grid=(NBLK,),
        in_specs=[
            pl.BlockSpec((EBP, W), lambda i: (i, 0)),
            pl.BlockSpec((EBP, W), lambda i: (i, 0)),
            pl.BlockSpec((EB, H), lambda i: (i, 0)),
            pl.BlockSpec((H, H), lambda i: (0, 0)),
            pl.BlockSpec((1, H), lambda i: (0, 0)),
            pl.BlockSpec((1, H), lambda i: (0, 0)),
            pl.BlockSpec((1, H), lambda i: (0, 0)),
        ],
        out_specs=(pl.BlockSpec((EB, H), lambda i: (i, 0)),
                   pl.BlockSpec((EBP, W), lambda i: (i, 0))),
        out_shape=(jax.ShapeDtypeStruct((E, H), jnp.float32),
                   jax.ShapeDtypeStruct((EP, W), jnp.float32)),
    )(g1_p, g2_p, ee, wc, bc, eg, eb)


def _tc_node_update_body(h_ref, ax_ref, acc_ref, g_ref, b_ref, o_ref):
    acc = acc_ref[:]
    num = acc[:, 0:H]
    den = acc[:, H:W]
    agg = num / (den + 1e-6)
    o_ref[:] = h_ref[:] + jnp.maximum(
        _ln(ax_ref[:] + agg, g_ref[:], b_ref[:]), 0.0)


def _node_update(h, ax, acc, g, b):
    return pl.pallas_call(
        _tc_node_update_body,
        out_shape=jax.ShapeDtypeStruct((N, H), jnp.float32),
    )(h, ax, acc, g, b)


def _tc_pred_node_body(h_ref, wa_ref, wb_ref, tp_o):
    h = h_ref[:]
    p1 = jnp.dot(h, wa_ref[:], preferred_element_type=jnp.float32)
    p2 = jnp.dot(h, wb_ref[:], preferred_element_type=jnp.float32)
    tp_o[:] = jnp.concatenate([p1, p2], axis=1)


def _pred_node(h, w1a, w1b):
    return pl.pallas_call(
        _tc_pred_node_body,
        out_shape=jax.ShapeDtypeStruct((N, W), jnp.float32),
    )(h, w1a, w1b)


def _tc_pred_final_body(gp1_ref, gp2_ref, ee_ref, w1c_ref, bp1_ref, wp2_ref,
                        bp2_ref, o_ref):
    z = jnp.maximum(
        gp1_ref[0:EB, 0:H] + gp2_ref[0:EB, H:W]
        + jnp.dot(ee_ref[:], w1c_ref[:], preferred_element_type=jnp.float32)
        + bp1_ref[:], 0.0)
    o_ref[:] = jnp.dot(z, wp2_ref[:], preferred_element_type=jnp.float32) + bp2_ref[:]


def _pred_final(gp1_p, gp2_p, ee, w1c, bp1, wp2, bp2):
    return pl.pallas_call(
        _tc_pred_final_body,
        grid=(NBLK,),
        in_specs=[
            pl.BlockSpec((EBP, W), lambda i: (i, 0)),
            pl.BlockSpec((EBP, W), lambda i: (i, 0)),
            pl.BlockSpec((EB, H), lambda i: (i, 0)),
            pl.BlockSpec((H, H), lambda i: (0, 0)),
            pl.BlockSpec((1, H), lambda i: (0, 0)),
            pl.BlockSpec((H, 1), lambda i: (0, 0)),
            pl.BlockSpec((1, 1), lambda i: (0, 0)),
        ],
        out_specs=pl.BlockSpec((EB, 1), lambda i: (i, 0)),
        out_shape=jax.ShapeDtypeStruct((E, 1), jnp.float32),
    )(gp1_p, gp2_p, ee, w1c, bp1, wp2, bp2)


def _pad_idx(v):
    v2 = v.reshape(NBLK, EB)
    v2 = jnp.pad(v2, ((0, 0), (0, EBP - EB)))
    return v2.reshape(EP // IB, IB // 128, 128)


def kernel(x, e, Wn1, bn1, Wn2, bn2, We1, be1, We2, be2, W_gnn, b_gnn,
           ln_ng, ln_nb, ln_eg, ln_eb, Wp1, bp1, Wp2, bp2, edge_index):
    src2d = _pad_idx(edge_index[0].astype(jnp.int32))
    dst2d = _pad_idx(edge_index[1].astype(jnp.int32))
    zrows = jnp.zeros((ZR, W), jnp.float32)

    row = lambda v: v.reshape(1, -1)

    h = _node_enc(x, Wn1, row(bn1), Wn2, row(bn2))
    ee = _edge_enc(e, We1, row(be1), We2, row(be2))

    num_layers = W_gnn.shape[0]
    for l in range(num_layers):
        wa, wb, wc, wd, we = (W_gnn[l, k] for k in range(5))
        ba, bb, bc, bd, be_ = (row(b_gnn[l, k]) for k in range(5))
        ax, t1, t2 = _node_mats(h, wa, wb, wd, we, ba, bb, bd, be_)
        g1_p = _sc_gather(t1, src2d)
        g2_p = _sc_gather(t2, dst2d)
        ee, ms_p = _edge_update(g1_p, g2_p, ee, wc, bc,
                                row(ln_eg[l]), row(ln_eb[l]))
        acc = _sc_scatter(ms_p, dst2d, zrows)
        h = _node_update(h, ax, acc, row(ln_ng[l]), row(ln_nb[l]))

    tp = _pred_node(h, Wp1[0:H], Wp1[H:2 * H])
    gp1_p = _sc_gather(tp, src2d)
    gp2_p = _sc_gather(tp, dst2d)
    scores = _pred_final(gp1_p, gp2_p, ee, Wp1[2 * H:3 * H], row(bp1), Wp2,
                         row(bp2).reshape(1, 1))
    return scores


# speedup vs baseline: 2.0055x; 1.0176x over previous
import jax
import jax.numpy as jnp
from jax import lax
from jax.experimental import pallas as pl
from jax.experimental.pallas import tpu as pltpu
from jax.experimental.pallas import tpu_sc as plsc

N = 10000
E = 320000
H = 64
W = 2 * H
EB = 2000
EBP = 2048
NBLK = E // EB
EP = EBP * NBLK
NW = 32
IB = 1024
NCH = EP // (NW * IB)
CHG = 256
NH = N // 2
NACC = NH + 8
NPT = 312
ZR = 104

_MESH = dict(core_axis_name="c", subcore_axis_name="s", num_cores=2,
             num_subcores=16)


def _sc_gather2_body(t1_hbm, t2_hbm, src_hbm, dst_hbm, g1_hbm, g2_hbm,
                     idx0, idx1, buf0, buf1, sem_i, sem_g, sem_w):
    c = lax.axis_index("c")
    s = lax.axis_index("s")
    wid = s * 2 + c
    bufs = (buf0, buf1)
    idxs = (idx0, idx1)

    rpb = IB // CHG
    for tab, idx_hbm, out_hbm in ((t1_hbm, src_hbm, g1_hbm),
                                  (t2_hbm, dst_hbm, g2_hbm)):
        icps = [None] * NCH
        wcps = [None, None]
        icps[0] = pltpu.async_copy(idx_hbm.at[wid * NCH], idxs[0], sem_i)
        for r in range(rpb * NCH):
            b = r // rpb
            if r % rpb == 0:
                if b + 1 < NCH:
                    icps[b + 1] = pltpu.async_copy(
                        idx_hbm.at[wid * NCH + b + 1], idxs[(b + 1) % 2],
                        sem_i)
                icps[b].wait()
            if wcps[r % 2] is not None:
                wcps[r % 2].wait()
            gcps = [pltpu.async_copy(
                tab.at[idxs[b % 2].at[(r % rpb) * 2 + j]],
                bufs[r % 2].at[pl.ds(j * 128, 128)], sem_g)
                for j in range(CHG // 128)]
            for cp in gcps:
                cp.wait()
            e0 = (wid * NCH + b) * IB + (r % rpb) * CHG
            wcps[r % 2] = pltpu.async_copy(
                bufs[r % 2], out_hbm.at[pl.ds(e0, CHG)], sem_w)
        for cp in wcps:
            if cp is not None:
                cp.wait()


def _sc_gather2(t1, t2, src3d, dst3d):
    sds = jax.ShapeDtypeStruct((EP, W), jnp.float32)
    f = pl.kernel(
        _sc_gather2_body,
        out_type=(sds, sds),
        mesh=plsc.VectorSubcoreMesh(**_MESH),
        scratch_types=[
            pltpu.VMEM((IB // 128, 128), jnp.int32),
            pltpu.VMEM((IB // 128, 128), jnp.int32),
            pltpu.VMEM((CHG, W), jnp.float32),
            pltpu.VMEM((CHG, W), jnp.float32),
            pltpu.SemaphoreType.DMA,
            pltpu.SemaphoreType.DMA,
            pltpu.SemaphoreType.DMA,
        ],
    )
    return f(t1, t2, src3d, dst3d)


CHS = 256
NRS = 2 * NCH * (IB // CHS)


def _sc_scatter_body(ms_hbm, idx_hbm, z_hbm, acc_out,
                     idx0, idx1, sb0, sb1, acc_sh, sem_i, sem_l, sem_s):
    c = lax.axis_index("c")
    s = lax.axis_index("s")

    r0 = s * NPT
    for k in range(3):
        pltpu.sync_copy(z_hbm, acc_sh.at[pl.ds(r0 + k * ZR, ZR)])

    @pl.when(s == 15)
    def _():
        pltpu.sync_copy(z_hbm.at[pl.ds(0, 8)], acc_sh.at[pl.ds(16 * NPT, 8)])
    plsc.subcore_barrier()

    idxs = (idx0, idx1)
    sbufs = (sb0, sb1)
    rpb = IB // CHS
    nblk = 2 * NCH
    icps = [None] * nblk
    lcps = [None, None]
    scps = [[], []]
    icps[0] = pltpu.async_copy(idx_hbm.at[c, s * nblk], idxs[0], sem_i)
    lcps[0] = pltpu.async_copy(
        ms_hbm.at[pl.ds(s * nblk * IB, CHS)], sbufs[0], sem_l)
    for r in range(NRS):
        b = r // rpb
        if r % rpb == 0:
            if b + 1 < nblk:
                icps[b + 1] = pltpu.async_copy(
                    idx_hbm.at[c, s * nblk + b + 1], idxs[(b + 1) % 2],
                    sem_i)
            icps[b].wait()
        lcps[r % 2].wait()
        scps[r % 2] = [
            pltpu.async_copy(sbufs[r % 2].at[pl.ds(j * 128, 128)],
                             acc_sh.at[idxs[b % 2].at[(r % rpb) * 2 + j]],
                             sem_s, add=True)
            for j in range(CHS // 128)]
        if r + 1 < NRS:
            for cp in scps[(r + 1) % 2]:
                cp.wait()
            e0 = (s * nblk) * IB + (r + 1) * CHS
            lcps[(r + 1) % 2] = pltpu.async_copy(
                ms_hbm.at[pl.ds(e0, CHS)], sbufs[(r + 1) % 2], sem_l)
    for cps in scps:
        for cp in cps:
            cp.wait()

    plsc.subcore_barrier()
    pltpu.sync_copy(acc_sh.at[pl.ds(r0, NPT)],
                    acc_out.at[pl.ds(c * NH + r0, NPT)])

    @pl.when(s == 15)
    def _():
        pltpu.sync_copy(acc_sh.at[pl.ds(16 * NPT, 8)],
                        acc_out.at[pl.ds(c * NH + 16 * NPT, 8)])


def _sc_scatter(ms_p, idx4d, zrows):
    f = pl.kernel(
        _sc_scatter_body,
        out_type=jax.ShapeDtypeStruct((N, W), jnp.float32),
        mesh=plsc.VectorSubcoreMesh(**_MESH),
        scratch_types=[
            pltpu.VMEM((IB // 128, 128), jnp.int32),
            pltpu.VMEM((IB // 128, 128), jnp.int32),
            pltpu.VMEM((CHS, W), jnp.float32),
            pltpu.VMEM((CHS, W), jnp.float32),
            pltpu.VMEM_SHARED((NACC, W), jnp.float32),
            pltpu.SemaphoreType.DMA,
            pltpu.SemaphoreType.DMA,
            pltpu.SemaphoreType.DMA,
        ],
    )
    return f(ms_p, idx4d, zrows)


def _ln(v, g, b):
    mu = jnp.mean(v, axis=-1, keepdims=True)
    var = jnp.mean((v - mu) * (v - mu), axis=-1, keepdims=True)
    return g * (v - mu) * lax.rsqrt(var + 1e-5) + b


def _tc_enc_body(x_ref, w1_ref, b1_ref, w2_ref, b2_ref, o_ref):
    hh = jnp.maximum(
        jnp.dot(x_ref[:], w1_ref[:], preferred_element_type=jnp.float32)
        + b1_ref[:], 0.0)
    o_ref[:] = jnp.dot(hh, w2_ref[:], preferred_element_type=jnp.float32) + b2_ref[:]


def _node_enc(x, w1, b1, w2, b2):
    return pl.pallas_call(
        _tc_enc_body,
        out_shape=jax.ShapeDtypeStruct((N, H), jnp.float32),
    )(x, w1, b1, w2, b2)


def _edge_enc(e, w1, b1, w2, b2):
    d_edge = e.shape[1]
    return pl.pallas_call(
        _tc_enc_body,
        grid=(NBLK,),
        in_specs=[
            pl.BlockSpec((EB, d_edge), lambda i: (i, 0)),
            pl.BlockSpec((d_edge, H), lambda i: (0, 0)),
            pl.BlockSpec((1, H), lambda i: (0, 0)),
            pl.BlockSpec((H, H), lambda i: (0, 0)),
            pl.BlockSpec((1, H), lambda i: (0, 0)),
        ],
        out_specs=pl.BlockSpec((EB, H), lambda i: (i, 0)),
        out_shape=jax.ShapeDtypeStruct((E, H), jnp.float32),
    )(e, w1, b1, w2, b2)


def _tc_node_mats_body(h_ref, wa, wb, wd, we, ba, bb, bd, be,
                       ax_o, t1_o, t2_o):
    h = h_ref[:]
    ax_o[:] = jnp.dot(h, wa[:], preferred_element_type=jnp.float32) + ba[:]
    bx = jnp.dot(h, wb[:], preferred_element_type=jnp.float32) + bb[:]
    dx = jnp.dot(h, wd[:], preferred_element_type=jnp.float32) + bd[:]
    ex = jnp.dot(h, we[:], preferred_element_type=jnp.float32) + be[:]
    t1_o[:] = jnp.concatenate([dx, bx], axis=1)
    t2_o[:] = jnp.concatenate([ex, jnp.zeros((N, H), jnp.float32)], axis=1)


def _node_mats(h, wa, wb, wd, we, ba, bb, bd, be):
    return pl.pallas_call(
        _tc_node_mats_body,
        out_shape=(jax.ShapeDtypeStruct((N, H), jnp.float32),
                   jax.ShapeDtypeStruct((N, W), jnp.float32),
                   jax.ShapeDtypeStruct((N, W), jnp.float32)),
    )(h, wa, wb, wd, we, ba, bb, bd, be)


def _tc_edge_update_body(g1_ref, g2_ref, ee_ref, c_ref, bc_ref, eg_ref,
                         eb_ref, eeo_ref, ms_ref):
    ee = ee_ref[:]
    ce = jnp.dot(ee, c_ref[:], preferred_element_type=jnp.float32) + bc_ref[:]
    g1 = g1_ref[0:EB, :]
    epre = g1[:, 0:H] + g2_ref[0:EB, 0:H] + ce
    eeo_ref[:] = ee + jnp.maximum(_ln(epre, eg_ref[:], eb_ref[:]), 0.0)
    sig = jax.nn.sigmoid(epre)
    msg = sig * g1[:, H:W]
    ms_ref[0:EB, :] = jnp.concatenate([msg, sig], axis=1)
    ms_ref[EB:EBP, :] = jnp.zeros((EBP - EB, W), jnp.float32)


def _edge_update(g1_p, g2_p, ee, wc, bc, eg, eb):
    return pl.pallas_call(
        _tc_edge_update_body,
        grid=(NBLK,),
        in_specs=[
            pl.BlockSpec((EBP, W), lambda i: (i, 0)),
            pl.BlockSpec((EBP, W), lambda i: (i, 0)),
            pl.BlockSpec((EB, H), lambda i: (i, 0)),
            pl.BlockSpec((H, H), lambda i: (0, 0)),
            pl.BlockSpec((1, H), lambda i: (0, 0)),
            pl.BlockSpec((1, H), lambda i: (0, 0)),
            pl.BlockSpec((1, H), lambda i: (0, 0)),
        ],
        out_specs=(pl.BlockSpec((EB, H), lambda i: (i, 0)),
                   pl.BlockSpec((EBP, W), lambda i: (i, 0))),
        out_shape=(jax.ShapeDtypeStruct((E, H), jnp.float32),
                   jax.ShapeDtypeStruct((EP, W), jnp.float32)),
    )(g1_p, g2_p, ee, wc, bc, eg, eb)


def _tc_node_update_body(h_ref, ax_ref, acc_ref, g_ref, b_ref, o_ref):
    acc = acc_ref[:]
    num = acc[:, 0:H]
    den = acc[:, H:W]
    agg = num / (den + 1e-6)
    o_ref[:] = h_ref[:] + jnp.maximum(
        _ln(ax_ref[:] + agg, g_ref[:], b_ref[:]), 0.0)


def _node_update(h, ax, acc, g, b):
    return pl.pallas_call(
        _tc_node_update_body,
        out_shape=jax.ShapeDtypeStruct((N, H), jnp.float32),
    )(h, ax, acc, g, b)


def _tc_pred_node_body(h_ref, wa_ref, wb_ref, tp_o):
    h = h_ref[:]
    p1 = jnp.dot(h, wa_ref[:], preferred_element_type=jnp.float32)
    p2 = jnp.dot(h, wb_ref[:], preferred_element_type=jnp.float32)
    tp_o[:] = jnp.concatenate([p1, p2], axis=1)


def _pred_node(h, w1a, w1b):
    return pl.pallas_call(
        _tc_pred_node_body,
        out_shape=jax.ShapeDtypeStruct((N, W), jnp.float32),
    )(h, w1a, w1b)


def _tc_pred_final_body(gp1_ref, gp2_ref, ee_ref, w1c_ref, bp1_ref, wp2_ref,
                        bp2_ref, o_ref):
    z = jnp.maximum(
        gp1_ref[0:EB, 0:H] + gp2_ref[0:EB, H:W]
        + jnp.dot(ee_ref[:], w1c_ref[:], preferred_element_type=jnp.float32)
        + bp1_ref[:], 0.0)
    o_ref[:] = jnp.dot(z, wp2_ref[:], preferred_element_type=jnp.float32) + bp2_ref[:]


def _pred_final(gp1_p, gp2_p, ee, w1c, bp1, wp2, bp2):
    return pl.pallas_call(
        _tc_pred_final_body,
        grid=(NBLK,),
        in_specs=[
            pl.BlockSpec((EBP, W), lambda i: (i, 0)),
            pl.BlockSpec((EBP, W), lambda i: (i, 0)),
            pl.BlockSpec((EB, H), lambda i: (i, 0)),
            pl.BlockSpec((H, H), lambda i: (0, 0)),
            pl.BlockSpec((1, H), lambda i: (0, 0)),
            pl.BlockSpec((H, 1), lambda i: (0, 0)),
            pl.BlockSpec((1, 1), lambda i: (0, 0)),
        ],
        out_specs=pl.BlockSpec((EB, 1), lambda i: (i, 0)),
        out_shape=jax.ShapeDtypeStruct((E, 1), jnp.float32),
    )(gp1_p, gp2_p, ee, w1c, bp1, wp2, bp2)


def _pad_idx(v):
    v2 = v.reshape(NBLK, EB)
    v2 = jnp.pad(v2, ((0, 0), (0, EBP - EB)))
    return v2.reshape(EP // IB, IB // 128, 128)


def kernel(x, e, Wn1, bn1, Wn2, bn2, We1, be1, We2, be2, W_gnn, b_gnn,
           ln_ng, ln_nb, ln_eg, ln_eb, Wp1, bp1, Wp2, bp2, edge_index):
    src3d = _pad_idx(edge_index[0].astype(jnp.int32))
    dst3d = _pad_idx(edge_index[1].astype(jnp.int32))
    dst_lo = jnp.where(dst3d < NH, dst3d, NH)
    dst_hi = jnp.where(dst3d >= NH, dst3d - NH, NH)
    dst4d = jnp.stack([dst_lo, dst_hi])
    zrows = jnp.zeros((ZR, W), jnp.float32)

    row = lambda v: v.reshape(1, -1)

    h = _node_enc(x, Wn1, row(bn1), Wn2, row(bn2))
    ee = _edge_enc(e, We1, row(be1), We2, row(be2))

    num_layers = W_gnn.shape[0]
    for l in range(num_layers):
        wa, wb, wc, wd, we = (W_gnn[l, k] for k in range(5))
        ba, bb, bc, bd, be_ = (row(b_gnn[l, k]) for k in range(5))
        ax, t1, t2 = _node_mats(h, wa, wb, wd, we, ba, bb, bd, be_)
        g1_p, g2_p = _sc_gather2(t1, t2, src3d, dst3d)
        ee, ms_p = _edge_update(g1_p, g2_p, ee, wc, bc,
                                row(ln_eg[l]), row(ln_eb[l]))
        acc = _sc_scatter(ms_p, dst4d, zrows)
        h = _node_update(h, ax, acc, row(ln_ng[l]), row(ln_nb[l]))

    tp = _pred_node(h, Wp1[0:H], Wp1[H:2 * H])
    gp1_p, gp2_p = _sc_gather2(tp, tp, src3d, dst3d)
    scores = _pred_final(gp1_p, gp2_p, ee, Wp1[2 * H:3 * H], row(bp1), Wp2,
                         row(bp2).reshape(1, 1))
    return scores


# speedup vs baseline: 2.0068x; 1.0007x over previous
import jax
import jax.numpy as jnp
from jax import lax
from jax.experimental import pallas as pl
from jax.experimental.pallas import tpu as pltpu
from jax.experimental.pallas import tpu_sc as plsc

N = 10000
E = 320000
H = 64
W = 2 * H
EB = 2000
EBP = 2048
NBLK = E // EB
EP = EBP * NBLK
NW = 32
IB = 1024
NCH = EP // (NW * IB)
CHG = 256
NH = N // 2
NACC = NH + 8
NPT = 312
ZR = 104

_MESH = dict(core_axis_name="c", subcore_axis_name="s", num_cores=2,
             num_subcores=16)


def _sc_gather2_body(t1_hbm, t2_hbm, src_hbm, dst_hbm, g1_hbm, g2_hbm,
                     idx0, idx1, buf0, buf1, sem_i, sem_g, sem_w):
    c = lax.axis_index("c")
    s = lax.axis_index("s")
    wid = s * 2 + c
    bufs = (buf0, buf1)
    idxs = (idx0, idx1)

    rpb = IB // CHG
    for tab, idx_hbm, out_hbm in ((t1_hbm, src_hbm, g1_hbm),
                                  (t2_hbm, dst_hbm, g2_hbm)):
        icps = [None] * NCH
        wcps = [None, None]
        icps[0] = pltpu.async_copy(idx_hbm.at[wid * NCH], idxs[0], sem_i)
        for r in range(rpb * NCH):
            b = r // rpb
            if r % rpb == 0:
                if b + 1 < NCH:
                    icps[b + 1] = pltpu.async_copy(
                        idx_hbm.at[wid * NCH + b + 1], idxs[(b + 1) % 2],
                        sem_i)
                icps[b].wait()
            if wcps[r % 2] is not None:
                wcps[r % 2].wait()
            gcps = [pltpu.async_copy(
                tab.at[idxs[b % 2].at[(r % rpb) * 2 + j]],
                bufs[r % 2].at[pl.ds(j * 128, 128)], sem_g)
                for j in range(CHG // 128)]
            for cp in gcps:
                cp.wait()
            e0 = (wid * NCH + b) * IB + (r % rpb) * CHG
            wcps[r % 2] = pltpu.async_copy(
                bufs[r % 2], out_hbm.at[pl.ds(e0, CHG)], sem_w)
        for cp in wcps:
            if cp is not None:
                cp.wait()


def _sc_gather2(t1, t2, src3d, dst3d):
    sds = jax.ShapeDtypeStruct((EP, W), jnp.float32)
    f = pl.kernel(
        _sc_gather2_body,
        out_type=(sds, sds),
        mesh=plsc.VectorSubcoreMesh(**_MESH),
        scratch_types=[
            pltpu.VMEM((IB // 128, 128), jnp.int32),
            pltpu.VMEM((IB // 128, 128), jnp.int32),
            pltpu.VMEM((CHG, W), jnp.float32),
            pltpu.VMEM((CHG, W), jnp.float32),
            pltpu.SemaphoreType.DMA,
            pltpu.SemaphoreType.DMA,
            pltpu.SemaphoreType.DMA,
        ],
    )
    return f(t1, t2, src3d, dst3d)


CHS = 256
NRS = 2 * NCH * (IB // CHS)


def _sc_scatter_body(ms_hbm, idx_hbm, z_hbm, acc_out,
                     idx0, idx1, sb0, sb1, acc_sh, sem_i, sem_l, sem_s):
    c = lax.axis_index("c")
    s = lax.axis_index("s")

    r0 = s * NPT
    for k in range(3):
        pltpu.sync_copy(z_hbm, acc_sh.at[pl.ds(r0 + k * ZR, ZR)])

    @pl.when(s == 15)
    def _():
        pltpu.sync_copy(z_hbm.at[pl.ds(0, 8)], acc_sh.at[pl.ds(16 * NPT, 8)])
    plsc.subcore_barrier()

    idxs = (idx0, idx1)
    sbufs = (sb0, sb1)
    rpb = IB // CHS
    nblk = 2 * NCH
    icps = [None] * nblk
    lcps = [None, None]
    scps = [None, None]
    icps[0] = pltpu.async_copy(idx_hbm.at[c, s * nblk], idxs[0], sem_i)
    lcps[0] = pltpu.async_copy(
        ms_hbm.at[pl.ds(s * nblk * IB, CHS)], sbufs[0], sem_l)
    for r in range(NRS):
        b = r // rpb
        if r % rpb == 0:
            if b + 1 < nblk:
                icps[b + 1] = pltpu.async_copy(
                    idx_hbm.at[c, s * nblk + b + 1], idxs[(b + 1) % 2],
                    sem_i)
            icps[b].wait()
        lcps[r % 2].wait()
        if scps[(r + 1) % 2] is not None:
            scps[(r + 1) % 2].wait()
            scps[(r + 1) % 2] = None
        if r + 1 < NRS:
            e0 = (s * nblk) * IB + (r + 1) * CHS
            lcps[(r + 1) % 2] = pltpu.async_copy(
                ms_hbm.at[pl.ds(e0, CHS)], sbufs[(r + 1) % 2], sem_l)
        last = None
        for j in range(CHS // 128):
            if last is not None:
                last.wait()
            last = pltpu.async_copy(
                sbufs[r % 2].at[pl.ds(j * 128, 128)],
                acc_sh.at[idxs[b % 2].at[(r % rpb) * 2 + j]],
                sem_s, add=True)
        scps[r % 2] = last
    for cp in scps:
        if cp is not None:
            cp.wait()

    plsc.subcore_barrier()
    pltpu.sync_copy(acc_sh.at[pl.ds(r0, NPT)],
                    acc_out.at[pl.ds(c * NH + r0, NPT)])

    @pl.when(s == 15)
    def _():
        pltpu.sync_copy(acc_sh.at[pl.ds(16 * NPT, 8)],
                        acc_out.at[pl.ds(c * NH + 16 * NPT, 8)])


def _sc_scatter(ms_p, idx4d, zrows):
    f = pl.kernel(
        _sc_scatter_body,
        out_type=jax.ShapeDtypeStruct((N, W), jnp.float32),
        mesh=plsc.VectorSubcoreMesh(**_MESH),
        scratch_types=[
            pltpu.VMEM((IB // 128, 128), jnp.int32),
            pltpu.VMEM((IB // 128, 128), jnp.int32),
            pltpu.VMEM((CHS, W), jnp.float32),
            pltpu.VMEM((CHS, W), jnp.float32),
            pltpu.VMEM_SHARED((NACC, W), jnp.float32),
            pltpu.SemaphoreType.DMA,
            pltpu.SemaphoreType.DMA,
            pltpu.SemaphoreType.DMA,
        ],
    )
    return f(ms_p, idx4d, zrows)


def _ln(v, g, b):
    mu = jnp.mean(v, axis=-1, keepdims=True)
    var = jnp.mean((v - mu) * (v - mu), axis=-1, keepdims=True)
    return g * (v - mu) * lax.rsqrt(var + 1e-5) + b


def _tc_enc_body(x_ref, w1_ref, b1_ref, w2_ref, b2_ref, o_ref):
    hh = jnp.maximum(
        jnp.dot(x_ref[:], w1_ref[:], preferred_element_type=jnp.float32)
        + b1_ref[:], 0.0)
    o_ref[:] = jnp.dot(hh, w2_ref[:], preferred_element_type=jnp.float32) + b2_ref[:]


def _node_enc(x, w1, b1, w2, b2):
    return pl.pallas_call(
        _tc_enc_body,
        out_shape=jax.ShapeDtypeStruct((N, H), jnp.float32),
    )(x, w1, b1, w2, b2)


def _edge_enc(e, w1, b1, w2, b2):
    d_edge = e.shape[1]
    return pl.pallas_call(
        _tc_enc_body,
        grid=(NBLK,),
        in_specs=[
            pl.BlockSpec((EB, d_edge), lambda i: (i, 0)),
            pl.BlockSpec((d_edge, H), lambda i: (0, 0)),
            pl.BlockSpec((1, H), lambda i: (0, 0)),
            pl.BlockSpec((H, H), lambda i: (0, 0)),
            pl.BlockSpec((1, H), lambda i: (0, 0)),
        ],
        out_specs=pl.BlockSpec((EB, H), lambda i: (i, 0)),
        out_shape=jax.ShapeDtypeStruct((E, H), jnp.float32),
    )(e, w1, b1, w2, b2)


def _tc_node_mats_body(h_ref, wa, wb, wd, we, ba, bb, bd, be,
                       ax_o, t1_o, t2_o):
    h = h_ref[:]
    ax_o[:] = jnp.dot(h, wa[:], preferred_element_type=jnp.float32) + ba[:]
    bx = jnp.dot(h, wb[:], preferred_element_type=jnp.float32) + bb[:]
    dx = jnp.dot(h, wd[:], preferred_element_type=jnp.float32) + bd[:]
    ex = jnp.dot(h, we[:], preferred_element_type=jnp.float32) + be[:]
    t1_o[:] = jnp.concatenate([dx, bx], axis=1)
    t2_o[:] = jnp.concatenate([ex, jnp.zeros((N, H), jnp.float32)], axis=1)


def _node_mats(h, wa, wb, wd, we, ba, bb, bd, be):
    return pl.pallas_call(
        _tc_node_mats_body,
        out_shape=(jax.ShapeDtypeStruct((N, H), jnp.float32),
                   jax.ShapeDtypeStruct((N, W), jnp.float32),
                   jax.ShapeDtypeStruct((N, W), jnp.float32)),
    )(h, wa, wb, wd, we, ba, bb, bd, be)


def _tc_edge_update_body(g1_ref, g2_ref, ee_ref, c_ref, bc_ref, eg_ref,
                         eb_ref, eeo_ref, ms_ref):
    ee = ee_ref[:]
    ce = jnp.dot(ee, c_ref[:], preferred_element_type=jnp.float32) + bc_ref[:]
    g1 = g1_ref[0:EB, :]
    epre = g1[:, 0:H] + g2_ref[0:EB, 0:H] + ce
    eeo_ref[:] = ee + jnp.maximum(_ln(epre, eg_ref[:], eb_ref[:]), 0.0)
    sig = jax.nn.sigmoid(epre)
    msg = sig * g1[:, H:W]
    ms_ref[0:EB, :] = jnp.concatenate([msg, sig], axis=1)
    ms_ref[EB:EBP, :] = jnp.zeros((EBP - EB, W), jnp.float32)


def _edge_update(g1_p, g2_p, ee, wc, bc, eg, eb):
    return pl.pallas_call(
        _tc_edge_update_body,
        grid=(NBLK,),
        in_specs=[
            pl.BlockSpec((EBP, W), lambda i: (i, 0)),
            pl.BlockSpec((EBP, W), lambda i: (i, 0)),
            pl.BlockSpec((EB, H), lambda i: (i, 0)),
            pl.BlockSpec((H, H), lambda i: (0, 0)),
            pl.BlockSpec((1, H), lambda i: (0, 0)),
            pl.BlockSpec((1, H), lambda i: (0, 0)),
            pl.BlockSpec((1, H), lambda i: (0, 0)),
        ],
        out_specs=(pl.BlockSpec((EB, H), lambda i: (i, 0)),
                   pl.BlockSpec((EBP, W), lambda i: (i, 0))),
        out_shape=(jax.ShapeDtypeStruct((E, H), jnp.float32),
                   jax.ShapeDtypeStruct((EP, W), jnp.float32)),
    )(g1_p, g2_p, ee, wc, bc, eg, eb)


def _tc_node_update_body(h_ref, ax_ref, acc_ref, g_ref, b_ref, o_ref):
    acc = acc_ref[:]
    num = acc[:, 0:H]
    den = acc[:, H:W]
    agg = num / (den + 1e-6)
    o_ref[:] = h_ref[:] + jnp.maximum(
        _ln(ax_ref[:] + agg, g_ref[:], b_ref[:]), 0.0)


def _node_update(h, ax, acc, g, b):
    return pl.pallas_call(
        _tc_node_update_body,
        out_shape=jax.ShapeDtypeStruct((N, H), jnp.float32),
    )(h, ax, acc, g, b)


def _tc_pred_node_body(h_ref, wa_ref, wb_ref, tp_o):
    h = h_ref[:]
    p1 = jnp.dot(h, wa_ref[:], preferred_element_type=jnp.float32)
    p2 = jnp.dot(h, wb_ref[:], preferred_element_type=jnp.float32)
    tp_o[:] = jnp.concatenate([p1, p2], axis=1)


def _pred_node(h, w1a, w1b):
    return pl.pallas_call(
        _tc_pred_node_body,
        out_shape=jax.ShapeDtypeStruct((N, W), jnp.float32),
    )(h, w1a, w1b)


def _tc_pred_final_body(gp1_ref, gp2_ref, ee_ref, w1c_ref, bp1_ref, wp2_ref,
                        bp2_ref, o_ref):
    z = jnp.maximum(
        gp1_ref[0:EB, 0:H] + gp2_ref[0:EB, H:W]
        + jnp.dot(ee_ref[:], w1c_ref[:], preferred_element_type=jnp.float32)
        + bp1_ref[:], 0.0)
    o_ref[:] = jnp.dot(z, wp2_ref[:], preferred_element_type=jnp.float32) + bp2_ref[:]


def _pred_final(gp1_p, gp2_p, ee, w1c, bp1, wp2, bp2):
    return pl.pallas_call(
        _tc_pred_final_body,
        grid=(NBLK,),
        in_specs=[
            pl.BlockSpec((EBP, W), lambda i: (i, 0)),
            pl.BlockSpec((EBP, W), lambda i: (i, 0)),
            pl.BlockSpec((EB, H), lambda i: (i, 0)),
            pl.BlockSpec((H, H), lambda i: (0, 0)),
            pl.BlockSpec((1, H), lambda i: (0, 0)),
            pl.BlockSpec((H, 1), lambda i: (0, 0)),
            pl.BlockSpec((1, 1), lambda i: (0, 0)),
        ],
        out_specs=pl.BlockSpec((EB, 1), lambda i: (i, 0)),
        out_shape=jax.ShapeDtypeStruct((E, 1), jnp.float32),
    )(gp1_p, gp2_p, ee, w1c, bp1, wp2, bp2)


def _pad_idx(v):
    v2 = v.reshape(NBLK, EB)
    v2 = jnp.pad(v2, ((0, 0), (0, EBP - EB)))
    return v2.reshape(EP // IB, IB // 128, 128)


def kernel(x, e, Wn1, bn1, Wn2, bn2, We1, be1, We2, be2, W_gnn, b_gnn,
           ln_ng, ln_nb, ln_eg, ln_eb, Wp1, bp1, Wp2, bp2, edge_index):
    src3d = _pad_idx(edge_index[0].astype(jnp.int32))
    dst3d = _pad_idx(edge_index[1].astype(jnp.int32))
    dst_lo = jnp.where(dst3d < NH, dst3d, NH)
    dst_hi = jnp.where(dst3d >= NH, dst3d - NH, NH)
    dst4d = jnp.stack([dst_lo, dst_hi])
    zrows = jnp.zeros((ZR, W), jnp.float32)

    row = lambda v: v.reshape(1, -1)

    h = _node_enc(x, Wn1, row(bn1), Wn2, row(bn2))
    ee = _edge_enc(e, We1, row(be1), We2, row(be2))

    num_layers = W_gnn.shape[0]
    for l in range(num_layers):
        wa, wb, wc, wd, we = (W_gnn[l, k] for k in range(5))
        ba, bb, bc, bd, be_ = (row(b_gnn[l, k]) for k in range(5))
        ax, t1, t2 = _node_mats(h, wa, wb, wd, we, ba, bb, bd, be_)
        g1_p, g2_p = _sc_gather2(t1, t2, src3d, dst3d)
        ee, ms_p = _edge_update(g1_p, g2_p, ee, wc, bc,
                                row(ln_eg[l]), row(ln_eb[l]))
        acc = _sc_scatter(ms_p, dst4d, zrows)
        h = _node_update(h, ax, acc, row(ln_ng[l]), row(ln_nb[l]))

    tp = _pred_node(h, Wp1[0:H], Wp1[H:2 * H])
    gp1_p, gp2_p = _sc_gather2(tp, tp, src3d, dst3d)
    scores = _pred_final(gp1_p, gp2_p, ee, Wp1[2 * H:3 * H], row(bp1), Wp2,
                         row(bp2).reshape(1, 1))
    return scores


# speedup vs baseline: 4.3168x; 2.1510x over previous
import jax
import jax.numpy as jnp
from jax import lax
from jax.experimental import pallas as pl
from jax.experimental.pallas import tpu as pltpu
from jax.experimental.pallas import tpu_sc as plsc

N = 10000
E = 320000
H = 64
W = 2 * H
EB = 2000
EBP = 2048
NBLK = E // EB
EP = EBP * NBLK
NW = 32
IB = 1024
NCH = EP // (NW * IB)
CHG = 128
NH = N // 2
NACC = NH + 8
NPT = 312
ZR = 104

_MESH = dict(core_axis_name="c", subcore_axis_name="s", num_cores=2,
             num_subcores=16)


NST = 624


def _sc_gather2_body(t1_hbm, t2_hbm, src_hbm, dst_hbm, g1_hbm, g2_hbm,
                     idx0, idx1, buf0, buf1, tabsh, sem_i, sem_g, sem_w):
    c = lax.axis_index("c")
    s = lax.axis_index("s")
    wid = s * 2 + c
    bufs = (buf0, buf1)
    idxs = (idx0, idx1)

    rpb = IB // CHG
    t0 = s * NST
    for tab, idx_hbm, out_hbm in ((t1_hbm, src_hbm, g1_hbm),
                                  (t2_hbm, dst_hbm, g2_hbm)):
        pltpu.sync_copy(tab.at[pl.ds(t0, NST)], tabsh.at[pl.ds(t0, NST)])

        @pl.when(s == 15)
        def _():
            pltpu.sync_copy(tab.at[pl.ds(16 * NST, N - 16 * NST)],
                            tabsh.at[pl.ds(16 * NST, N - 16 * NST)])
        plsc.subcore_barrier()

        icps = [None] * NCH
        wcps = [None, None]
        icps[0] = pltpu.async_copy(idx_hbm.at[wid * NCH], idxs[0], sem_i)
        for r in range(rpb * NCH):
            b = r // rpb
            if r % rpb == 0:
                if b + 1 < NCH:
                    icps[b + 1] = pltpu.async_copy(
                        idx_hbm.at[wid * NCH + b + 1], idxs[(b + 1) % 2],
                        sem_i)
                icps[b].wait()
            if wcps[r % 2] is not None:
                wcps[r % 2].wait()
            gcps = [pltpu.async_copy(
                tabsh.at[idxs[b % 2].at[(r % rpb) * (CHG // 128) + j]],
                bufs[r % 2].at[pl.ds(j * 128, 128)], sem_g)
                for j in range(CHG // 128)]
            for cp in gcps:
                cp.wait()
            e0 = (wid * NCH + b) * IB + (r % rpb) * CHG
            wcps[r % 2] = pltpu.async_copy(
                bufs[r % 2], out_hbm.at[pl.ds(e0, CHG)], sem_w)
        for cp in wcps:
            if cp is not None:
                cp.wait()
        plsc.subcore_barrier()


def _sc_gather2(t1, t2, src3d, dst3d):
    sds = jax.ShapeDtypeStruct((EP, W), jnp.float32)
    f = pl.kernel(
        _sc_gather2_body,
        out_type=(sds, sds),
        mesh=plsc.VectorSubcoreMesh(**_MESH),
        scratch_types=[
            pltpu.VMEM((IB // 128, 128), jnp.int32),
            pltpu.VMEM((IB // 128, 128), jnp.int32),
            pltpu.VMEM((CHG, W), jnp.float32),
            pltpu.VMEM((CHG, W), jnp.float32),
            pltpu.VMEM_SHARED((N, W), jnp.float32),
            pltpu.SemaphoreType.DMA,
            pltpu.SemaphoreType.DMA,
            pltpu.SemaphoreType.DMA,
        ],
    )
    return f(t1, t2, src3d, dst3d)


CHS = 256
NRS = 2 * NCH * (IB // CHS)


def _sc_scatter_body(ms_hbm, idx_hbm, z_hbm, acc_out,
                     idx0, idx1, sb0, sb1, acc_sh, sem_i, sem_l, sem_s):
    c = lax.axis_index("c")
    s = lax.axis_index("s")

    r0 = s * NPT
    for k in range(3):
        pltpu.sync_copy(z_hbm, acc_sh.at[pl.ds(r0 + k * ZR, ZR)])

    @pl.when(s == 15)
    def _():
        pltpu.sync_copy(z_hbm.at[pl.ds(0, 8)], acc_sh.at[pl.ds(16 * NPT, 8)])
    plsc.subcore_barrier()

    idxs = (idx0, idx1)
    sbufs = (sb0, sb1)
    rpb = IB // CHS
    nblk = 2 * NCH
    icps = [None] * nblk
    lcps = [None, None]
    scps = [None, None]
    icps[0] = pltpu.async_copy(idx_hbm.at[c, s * nblk], idxs[0], sem_i)
    lcps[0] = pltpu.async_copy(
        ms_hbm.at[pl.ds(s * nblk * IB, CHS)], sbufs[0], sem_l)
    for r in range(NRS):
        b = r // rpb
        if r % rpb == 0:
            if b + 1 < nblk:
                icps[b + 1] = pltpu.async_copy(
                    idx_hbm.at[c, s * nblk + b + 1], idxs[(b + 1) % 2],
                    sem_i)
            icps[b].wait()
        lcps[r % 2].wait()
        if scps[(r + 1) % 2] is not None:
            scps[(r + 1) % 2].wait()
            scps[(r + 1) % 2] = None
        if r + 1 < NRS:
            e0 = (s * nblk) * IB + (r + 1) * CHS
            lcps[(r + 1) % 2] = pltpu.async_copy(
                ms_hbm.at[pl.ds(e0, CHS)], sbufs[(r + 1) % 2], sem_l)
        last = None
        for j in range(CHS // 128):
            if last is not None:
                last.wait()
            last = pltpu.async_copy(
                sbufs[r % 2].at[pl.ds(j * 128, 128)],
                acc_sh.at[idxs[b % 2].at[(r % rpb) * 2 + j]],
                sem_s, add=True)
        scps[r % 2] = last
    for cp in scps:
        if cp is not None:
            cp.wait()

    plsc.subcore_barrier()
    pltpu.sync_copy(acc_sh.at[pl.ds(r0, NPT)],
                    acc_out.at[pl.ds(c * NH + r0, NPT)])

    @pl.when(s == 15)
    def _():
        pltpu.sync_copy(acc_sh.at[pl.ds(16 * NPT, 8)],
                        acc_out.at[pl.ds(c * NH + 16 * NPT, 8)])


def _sc_scatter(ms_p, idx4d, zrows):
    f = pl.kernel(
        _sc_scatter_body,
        out_type=jax.ShapeDtypeStruct((N, W), jnp.float32),
        mesh=plsc.VectorSubcoreMesh(**_MESH),
        scratch_types=[
            pltpu.VMEM((IB // 128, 128), jnp.int32),
            pltpu.VMEM((IB // 128, 128), jnp.int32),
            pltpu.VMEM((CHS, W), jnp.float32),
            pltpu.VMEM((CHS, W), jnp.float32),
            pltpu.VMEM_SHARED((NACC, W), jnp.float32),
            pltpu.SemaphoreType.DMA,
            pltpu.SemaphoreType.DMA,
            pltpu.SemaphoreType.DMA,
        ],
    )
    return f(ms_p, idx4d, zrows)


def _ln(v, g, b):
    mu = jnp.mean(v, axis=-1, keepdims=True)
    var = jnp.mean((v - mu) * (v - mu), axis=-1, keepdims=True)
    return g * (v - mu) * lax.rsqrt(var + 1e-5) + b


def _tc_enc_body(x_ref, w1_ref, b1_ref, w2_ref, b2_ref, o_ref):
    hh = jnp.maximum(
        jnp.dot(x_ref[:], w1_ref[:], preferred_element_type=jnp.float32)
        + b1_ref[:], 0.0)
    o_ref[:] = jnp.dot(hh, w2_ref[:], preferred_element_type=jnp.float32) + b2_ref[:]


def _node_enc(x, w1, b1, w2, b2):
    return pl.pallas_call(
        _tc_enc_body,
        out_shape=jax.ShapeDtypeStruct((N, H), jnp.float32),
    )(x, w1, b1, w2, b2)


def _edge_enc(e, w1, b1, w2, b2):
    d_edge = e.shape[1]
    return pl.pallas_call(
        _tc_enc_body,
        grid=(NBLK,),
        in_specs=[
            pl.BlockSpec((EB, d_edge), lambda i: (i, 0)),
            pl.BlockSpec((d_edge, H), lambda i: (0, 0)),
            pl.BlockSpec((1, H), lambda i: (0, 0)),
            pl.BlockSpec((H, H), lambda i: (0, 0)),
            pl.BlockSpec((1, H), lambda i: (0, 0)),
        ],
        out_specs=pl.BlockSpec((EB, H), lambda i: (i, 0)),
        out_shape=jax.ShapeDtypeStruct((E, H), jnp.float32),
    )(e, w1, b1, w2, b2)


def _tc_node_mats_body(h_ref, wa, wb, wd, we, ba, bb, bd, be,
                       ax_o, t1_o, t2_o):
    h = h_ref[:]
    ax_o[:] = jnp.dot(h, wa[:], preferred_element_type=jnp.float32) + ba[:]
    bx = jnp.dot(h, wb[:], preferred_element_type=jnp.float32) + bb[:]
    dx = jnp.dot(h, wd[:], preferred_element_type=jnp.float32) + bd[:]
    ex = jnp.dot(h, we[:], preferred_element_type=jnp.float32) + be[:]
    t1_o[:] = jnp.concatenate([dx, bx], axis=1)
    t2_o[:] = jnp.concatenate([ex, jnp.zeros((N, H), jnp.float32)], axis=1)


def _node_mats(h, wa, wb, wd, we, ba, bb, bd, be):
    return pl.pallas_call(
        _tc_node_mats_body,
        out_shape=(jax.ShapeDtypeStruct((N, H), jnp.float32),
                   jax.ShapeDtypeStruct((N, W), jnp.float32),
                   jax.ShapeDtypeStruct((N, W), jnp.float32)),
    )(h, wa, wb, wd, we, ba, bb, bd, be)


def _tc_edge_update_body(g1_ref, g2_ref, ee_ref, c_ref, bc_ref, eg_ref,
                         eb_ref, eeo_ref, ms_ref):
    ee = ee_ref[:]
    ce = jnp.dot(ee, c_ref[:], preferred_element_type=jnp.float32) + bc_ref[:]
    g1 = g1_ref[0:EB, :]
    epre = g1[:, 0:H] + g2_ref[0:EB, 0:H] + ce
    eeo_ref[:] = ee + jnp.maximum(_ln(epre, eg_ref[:], eb_ref[:]), 0.0)
    sig = jax.nn.sigmoid(epre)
    msg = sig * g1[:, H:W]
    ms_ref[0:EB, :] = jnp.concatenate([msg, sig], axis=1)
    ms_ref[EB:EBP, :] = jnp.zeros((EBP - EB, W), jnp.float32)


def _edge_update(g1_p, g2_p, ee, wc, bc, eg, eb):
    return pl.pallas_call(
        _tc_edge_update_body,
        grid=(NBLK,),
        in_specs=[
            pl.BlockSpec((EBP, W), lambda i: (i, 0)),
            pl.BlockSpec((EBP, W), lambda i: (i, 0)),
            pl.BlockSpec((EB, H), lambda i: (i, 0)),
            pl.BlockSpec((H, H), lambda i: (0, 0)),
            pl.BlockSpec((1, H), lambda i: (0, 0)),
            pl.BlockSpec((1, H), lambda i: (0, 0)),
            pl.BlockSpec((1, H), lambda i: (0, 0)),
        ],
        out_specs=(pl.BlockSpec((EB, H), lambda i: (i, 0)),
                   pl.BlockSpec((EBP, W), lambda i: (i, 0))),
        out_shape=(jax.ShapeDtypeStruct((E, H), jnp.float32),
                   jax.ShapeDtypeStruct((EP, W), jnp.float32)),
    )(g1_p, g2_p, ee, wc, bc, eg, eb)


def _tc_node_update_body(h_ref, ax_ref, acc_ref, g_ref, b_ref, o_ref):
    acc = acc_ref[:]
    num = acc[:, 0:H]
    den = acc[:, H:W]
    agg = num / (den + 1e-6)
    o_ref[:] = h_ref[:] + jnp.maximum(
        _ln(ax_ref[:] + agg, g_ref[:], b_ref[:]), 0.0)


def _node_update(h, ax, acc, g, b):
    return pl.pallas_call(
        _tc_node_update_body,
        out_shape=jax.ShapeDtypeStruct((N, H), jnp.float32),
    )(h, ax, acc, g, b)


def _tc_pred_node_body(h_ref, wa_ref, wb_ref, tp_o):
    h = h_ref[:]
    p1 = jnp.dot(h, wa_ref[:], preferred_element_type=jnp.float32)
    p2 = jnp.dot(h, wb_ref[:], preferred_element_type=jnp.float32)
    tp_o[:] = jnp.concatenate([p1, p2], axis=1)


def _pred_node(h, w1a, w1b):
    return pl.pallas_call(
        _tc_pred_node_body,
        out_shape=jax.ShapeDtypeStruct((N, W), jnp.float32),
    )(h, w1a, w1b)


def _tc_pred_final_body(gp1_ref, gp2_ref, ee_ref, w1c_ref, bp1_ref, wp2_ref,
                        bp2_ref, o_ref):
    z = jnp.maximum(
        gp1_ref[0:EB, 0:H] + gp2_ref[0:EB, H:W]
        + jnp.dot(ee_ref[:], w1c_ref[:], preferred_element_type=jnp.float32)
        + bp1_ref[:], 0.0)
    o_ref[:] = jnp.dot(z, wp2_ref[:], preferred_element_type=jnp.float32) + bp2_ref[:]


def _pred_final(gp1_p, gp2_p, ee, w1c, bp1, wp2, bp2):
    return pl.pallas_call(
        _tc_pred_final_body,
        grid=(NBLK,),
        in_specs=[
            pl.BlockSpec((EBP, W), lambda i: (i, 0)),
            pl.BlockSpec((EBP, W), lambda i: (i, 0)),
            pl.BlockSpec((EB, H), lambda i: (i, 0)),
            pl.BlockSpec((H, H), lambda i: (0, 0)),
            pl.BlockSpec((1, H), lambda i: (0, 0)),
            pl.BlockSpec((H, 1), lambda i: (0, 0)),
            pl.BlockSpec((1, 1), lambda i: (0, 0)),
        ],
        out_specs=pl.BlockSpec((EB, 1), lambda i: (i, 0)),
        out_shape=jax.ShapeDtypeStruct((E, 1), jnp.float32),
    )(gp1_p, gp2_p, ee, w1c, bp1, wp2, bp2)


def _pad_idx(v):
    v2 = v.reshape(NBLK, EB)
    v2 = jnp.pad(v2, ((0, 0), (0, EBP - EB)))
    return v2.reshape(EP // IB, IB // 128, 128)


def kernel(x, e, Wn1, bn1, Wn2, bn2, We1, be1, We2, be2, W_gnn, b_gnn,
           ln_ng, ln_nb, ln_eg, ln_eb, Wp1, bp1, Wp2, bp2, edge_index):
    src3d = _pad_idx(edge_index[0].astype(jnp.int32))
    dst3d = _pad_idx(edge_index[1].astype(jnp.int32))
    dst_lo = jnp.where(dst3d < NH, dst3d, NH)
    dst_hi = jnp.where(dst3d >= NH, dst3d - NH, NH)
    dst4d = jnp.stack([dst_lo, dst_hi])
    zrows = jnp.zeros((ZR, W), jnp.float32)

    row = lambda v: v.reshape(1, -1)

    h = _node_enc(x, Wn1, row(bn1), Wn2, row(bn2))
    ee = _edge_enc(e, We1, row(be1), We2, row(be2))

    num_layers = W_gnn.shape[0]
    for l in range(num_layers):
        wa, wb, wc, wd, we = (W_gnn[l, k] for k in range(5))
        ba, bb, bc, bd, be_ = (row(b_gnn[l, k]) for k in range(5))
        ax, t1, t2 = _node_mats(h, wa, wb, wd, we, ba, bb, bd, be_)
        g1_p, g2_p = _sc_gather2(t1, t2, src3d, dst3d)
        ee, ms_p = _edge_update(g1_p, g2_p, ee, wc, bc,
                                row(ln_eg[l]), row(ln_eb[l]))
        acc = _sc_scatter(ms_p, dst4d, zrows)
        h = _node_update(h, ax, acc, row(ln_ng[l]), row(ln_nb[l]))

    tp = _pred_node(h, Wp1[0:H], Wp1[H:2 * H])
    gp1_p, gp2_p = _sc_gather2(tp, tp, src3d, dst3d)
    scores = _pred_final(gp1_p, gp2_p, ee, Wp1[2 * H:3 * H], row(bp1), Wp2,
                         row(bp2).reshape(1, 1))
    return scores


# speedup vs baseline: 5.0315x; 1.1656x over previous
import jax
import jax.numpy as jnp
from jax import lax
from jax.experimental import pallas as pl
from jax.experimental.pallas import tpu as pltpu
from jax.experimental.pallas import tpu_sc as plsc

N = 10000
E = 320000
H = 64
W = 2 * H
EB = 2000
EBP = 2048
NBLK = E // EB
EP = EBP * NBLK
NW = 32
IB = 1024
NCH = EP // (NW * IB)
CHG = 128
NPT = 624
ZR = 208

_MESH = dict(core_axis_name="c", subcore_axis_name="s", num_cores=2,
             num_subcores=16)


NST = 624


def _sc_gather2_body(t1_hbm, t2_hbm, src_hbm, dst_hbm, g1_hbm, g2_hbm,
                     idx0, idx1, buf0, buf1, tabsh, sem_i, sem_g, sem_w):
    c = lax.axis_index("c")
    s = lax.axis_index("s")
    wid = s * 2 + c
    bufs = (buf0, buf1)
    idxs = (idx0, idx1)

    rpb = IB // CHG
    t0 = s * NST
    for tab, idx_hbm, out_hbm in ((t1_hbm, src_hbm, g1_hbm),
                                  (t2_hbm, dst_hbm, g2_hbm)):
        pltpu.sync_copy(tab.at[pl.ds(t0, NST)], tabsh.at[pl.ds(t0, NST)])

        @pl.when(s == 15)
        def _():
            pltpu.sync_copy(tab.at[pl.ds(16 * NST, N - 16 * NST)],
                            tabsh.at[pl.ds(16 * NST, N - 16 * NST)])
        plsc.subcore_barrier()

        icps = [None] * NCH
        wcps = [None, None]
        icps[0] = pltpu.async_copy(idx_hbm.at[wid * NCH], idxs[0], sem_i)
        for r in range(rpb * NCH):
            b = r // rpb
            if r % rpb == 0:
                if b + 1 < NCH:
                    icps[b + 1] = pltpu.async_copy(
                        idx_hbm.at[wid * NCH + b + 1], idxs[(b + 1) % 2],
                        sem_i)
                icps[b].wait()
            if wcps[r % 2] is not None:
                wcps[r % 2].wait()
            gcps = [pltpu.async_copy(
                tabsh.at[idxs[b % 2].at[(r % rpb) * (CHG // 128) + j]],
                bufs[r % 2].at[pl.ds(j * 128, 128)], sem_g)
                for j in range(CHG // 128)]
            for cp in gcps:
                cp.wait()
            e0 = (wid * NCH + b) * IB + (r % rpb) * CHG
            wcps[r % 2] = pltpu.async_copy(
                bufs[r % 2], out_hbm.at[pl.ds(e0, CHG)], sem_w)
        for cp in wcps:
            if cp is not None:
                cp.wait()
        plsc.subcore_barrier()


def _sc_gather2(t1, t2, src3d, dst3d):
    sds = jax.ShapeDtypeStruct((EP, W), jnp.float32)
    f = pl.kernel(
        _sc_gather2_body,
        out_type=(sds, sds),
        mesh=plsc.VectorSubcoreMesh(**_MESH),
        scratch_types=[
            pltpu.VMEM((IB // 128, 128), jnp.int32),
            pltpu.VMEM((IB // 128, 128), jnp.int32),
            pltpu.VMEM((CHG, W), jnp.float32),
            pltpu.VMEM((CHG, W), jnp.float32),
            pltpu.VMEM_SHARED((N, W), jnp.float32),
            pltpu.SemaphoreType.DMA,
            pltpu.SemaphoreType.DMA,
            pltpu.SemaphoreType.DMA,
        ],
    )
    return f(t1, t2, src3d, dst3d)


CHS = 128
NRS = NCH * (IB // CHS)


def _sc_scatter_body(ms_hbm, idx_hbm, z_hbm, acc_out,
                     idx0, idx1, sb0, sb1, acc_sh, sem_i, sem_l, sem_s):
    c = lax.axis_index("c")
    s = lax.axis_index("s")
    wid = s * 2 + c

    r0 = s * NPT
    for k in range(3):
        pltpu.sync_copy(z_hbm, acc_sh.at[pl.ds(r0 + k * ZR, ZR)])

    @pl.when(s == 15)
    def _():
        pltpu.sync_copy(z_hbm.at[pl.ds(0, 16)], acc_sh.at[pl.ds(16 * NPT, 16)])
    plsc.subcore_barrier()

    idxs = (idx0, idx1)
    sbufs = (sb0, sb1)
    rpb = IB // CHS
    icps = [None] * NCH
    lcps = [None, None]
    scps = [None, None]
    icps[0] = pltpu.async_copy(idx_hbm.at[wid * NCH], idxs[0], sem_i)
    lcps[0] = pltpu.async_copy(
        ms_hbm.at[pl.ds(wid * NCH * IB, CHS)], sbufs[0], sem_l)
    for r in range(NRS):
        b = r // rpb
        if r % rpb == 0:
            if b + 1 < NCH:
                icps[b + 1] = pltpu.async_copy(
                    idx_hbm.at[wid * NCH + b + 1], idxs[(b + 1) % 2],
                    sem_i)
            icps[b].wait()
        lcps[r % 2].wait()
        if scps[(r + 1) % 2] is not None:
            scps[(r + 1) % 2].wait()
            scps[(r + 1) % 2] = None
        if r + 1 < NRS:
            e0 = wid * NCH * IB + (r + 1) * CHS
            lcps[(r + 1) % 2] = pltpu.async_copy(
                ms_hbm.at[pl.ds(e0, CHS)], sbufs[(r + 1) % 2], sem_l)
        scps[r % 2] = pltpu.async_copy(
            sbufs[r % 2], acc_sh.at[idxs[b % 2].at[r % rpb]],
            sem_s, add=True)
    for cp in scps:
        if cp is not None:
            cp.wait()

    plsc.subcore_barrier()
    pltpu.sync_copy(acc_sh.at[pl.ds(r0, NPT)], acc_out.at[c, pl.ds(r0, NPT)])

    @pl.when(s == 15)
    def _():
        pltpu.sync_copy(acc_sh.at[pl.ds(16 * NPT, 16)],
                        acc_out.at[c, pl.ds(16 * NPT, 16)])


def _sc_scatter(ms_p, idx3d, zrows):
    f = pl.kernel(
        _sc_scatter_body,
        out_type=jax.ShapeDtypeStruct((2, N, W), jnp.float32),
        mesh=plsc.VectorSubcoreMesh(**_MESH),
        scratch_types=[
            pltpu.VMEM((IB // 128, 128), jnp.int32),
            pltpu.VMEM((IB // 128, 128), jnp.int32),
            pltpu.VMEM((CHS, W), jnp.float32),
            pltpu.VMEM((CHS, W), jnp.float32),
            pltpu.VMEM_SHARED((N, W), jnp.float32),
            pltpu.SemaphoreType.DMA,
            pltpu.SemaphoreType.DMA,
            pltpu.SemaphoreType.DMA,
        ],
    )
    return f(ms_p, idx3d, zrows)


def _ln(v, g, b):
    mu = jnp.mean(v, axis=-1, keepdims=True)
    var = jnp.mean((v - mu) * (v - mu), axis=-1, keepdims=True)
    return g * (v - mu) * lax.rsqrt(var + 1e-5) + b


def _tc_enc_body(x_ref, w1_ref, b1_ref, w2_ref, b2_ref, o_ref):
    hh = jnp.maximum(
        jnp.dot(x_ref[:], w1_ref[:], preferred_element_type=jnp.float32)
        + b1_ref[:], 0.0)
    o_ref[:] = jnp.dot(hh, w2_ref[:], preferred_element_type=jnp.float32) + b2_ref[:]


def _node_enc(x, w1, b1, w2, b2):
    return pl.pallas_call(
        _tc_enc_body,
        out_shape=jax.ShapeDtypeStruct((N, H), jnp.float32),
    )(x, w1, b1, w2, b2)


def _edge_enc(e, w1, b1, w2, b2):
    d_edge = e.shape[1]
    return pl.pallas_call(
        _tc_enc_body,
        grid=(NBLK,),
        in_specs=[
            pl.BlockSpec((EB, d_edge), lambda i: (i, 0)),
            pl.BlockSpec((d_edge, H), lambda i: (0, 0)),
            pl.BlockSpec((1, H), lambda i: (0, 0)),
            pl.BlockSpec((H, H), lambda i: (0, 0)),
            pl.BlockSpec((1, H), lambda i: (0, 0)),
        ],
        out_specs=pl.BlockSpec((EB, H), lambda i: (i, 0)),
        out_shape=jax.ShapeDtypeStruct((E, H), jnp.float32),
    )(e, w1, b1, w2, b2)


def _tc_node_mats_body(h_ref, wa, wb, wd, we, ba, bb, bd, be,
                       ax_o, t1_o, t2_o):
    h = h_ref[:]
    ax_o[:] = jnp.dot(h, wa[:], preferred_element_type=jnp.float32) + ba[:]
    bx = jnp.dot(h, wb[:], preferred_element_type=jnp.float32) + bb[:]
    dx = jnp.dot(h, wd[:], preferred_element_type=jnp.float32) + bd[:]
    ex = jnp.dot(h, we[:], preferred_element_type=jnp.float32) + be[:]
    t1_o[:] = jnp.concatenate([dx, bx], axis=1)
    t2_o[:] = jnp.concatenate([ex, jnp.zeros((N, H), jnp.float32)], axis=1)


def _node_mats(h, wa, wb, wd, we, ba, bb, bd, be):
    return pl.pallas_call(
        _tc_node_mats_body,
        out_shape=(jax.ShapeDtypeStruct((N, H), jnp.float32),
                   jax.ShapeDtypeStruct((N, W), jnp.float32),
                   jax.ShapeDtypeStruct((N, W), jnp.float32)),
    )(h, wa, wb, wd, we, ba, bb, bd, be)


def _tc_edge_update_body(g1_ref, g2_ref, ee_ref, c_ref, bc_ref, eg_ref,
                         eb_ref, eeo_ref, ms_ref):
    ee = ee_ref[:]
    ce = jnp.dot(ee, c_ref[:], preferred_element_type=jnp.float32) + bc_ref[:]
    g1 = g1_ref[0:EB, :]
    epre = g1[:, 0:H] + g2_ref[0:EB, 0:H] + ce
    eeo_ref[:] = ee + jnp.maximum(_ln(epre, eg_ref[:], eb_ref[:]), 0.0)
    sig = jax.nn.sigmoid(epre)
    msg = sig * g1[:, H:W]
    ms_ref[0:EB, :] = jnp.concatenate([msg, sig], axis=1)
    ms_ref[EB:EBP, :] = jnp.zeros((EBP - EB, W), jnp.float32)


def _edge_update(g1_p, g2_p, ee, wc, bc, eg, eb):
    return pl.pallas_call(
        _tc_edge_update_body,
        grid=(NBLK,),
        in_specs=[
            pl.BlockSpec((EBP, W), lambda i: (i, 0)),
            pl.BlockSpec((EBP, W), lambda i: (i, 0)),
            pl.BlockSpec((EB, H), lambda i: (i, 0)),
            pl.BlockSpec((H, H), lambda i: (0, 0)),
            pl.BlockSpec((1, H), lambda i: (0, 0)),
            pl.BlockSpec((1, H), lambda i: (0, 0)),
            pl.BlockSpec((1, H), lambda i: (0, 0)),
        ],
        out_specs=(pl.BlockSpec((EB, H), lambda i: (i, 0)),
                   pl.BlockSpec((EBP, W), lambda i: (i, 0))),
        out_shape=(jax.ShapeDtypeStruct((E, H), jnp.float32),
                   jax.ShapeDtypeStruct((EP, W), jnp.float32)),
    )(g1_p, g2_p, ee, wc, bc, eg, eb)


def _tc_node_update_body(h_ref, ax_ref, acc_ref, g_ref, b_ref, o_ref):
    acc = acc_ref[0] + acc_ref[1]
    num = acc[:, 0:H]
    den = acc[:, H:W]
    agg = num / (den + 1e-6)
    o_ref[:] = h_ref[:] + jnp.maximum(
        _ln(ax_ref[:] + agg, g_ref[:], b_ref[:]), 0.0)


def _node_update(h, ax, acc, g, b):
    return pl.pallas_call(
        _tc_node_update_body,
        out_shape=jax.ShapeDtypeStruct((N, H), jnp.float32),
    )(h, ax, acc, g, b)


def _tc_pred_node_body(h_ref, wa_ref, wb_ref, tp_o):
    h = h_ref[:]
    p1 = jnp.dot(h, wa_ref[:], preferred_element_type=jnp.float32)
    p2 = jnp.dot(h, wb_ref[:], preferred_element_type=jnp.float32)
    tp_o[:] = jnp.concatenate([p1, p2], axis=1)


def _pred_node(h, w1a, w1b):
    return pl.pallas_call(
        _tc_pred_node_body,
        out_shape=jax.ShapeDtypeStruct((N, W), jnp.float32),
    )(h, w1a, w1b)


def _tc_pred_final_body(gp1_ref, gp2_ref, ee_ref, w1c_ref, bp1_ref, wp2_ref,
                        bp2_ref, o_ref):
    z = jnp.maximum(
        gp1_ref[0:EB, 0:H] + gp2_ref[0:EB, H:W]
        + jnp.dot(ee_ref[:], w1c_ref[:], preferred_element_type=jnp.float32)
        + bp1_ref[:], 0.0)
    o_ref[:] = jnp.dot(z, wp2_ref[:], preferred_element_type=jnp.float32) + bp2_ref[:]


def _pred_final(gp1_p, gp2_p, ee, w1c, bp1, wp2, bp2):
    return pl.pallas_call(
        _tc_pred_final_body,
        grid=(NBLK,),
        in_specs=[
            pl.BlockSpec((EBP, W), lambda i: (i, 0)),
            pl.BlockSpec((EBP, W), lambda i: (i, 0)),
            pl.BlockSpec((EB, H), lambda i: (i, 0)),
            pl.BlockSpec((H, H), lambda i: (0, 0)),
            pl.BlockSpec((1, H), lambda i: (0, 0)),
            pl.BlockSpec((H, 1), lambda i: (0, 0)),
            pl.BlockSpec((1, 1), lambda i: (0, 0)),
        ],
        out_specs=pl.BlockSpec((EB, 1), lambda i: (i, 0)),
        out_shape=jax.ShapeDtypeStruct((E, 1), jnp.float32),
    )(gp1_p, gp2_p, ee, w1c, bp1, wp2, bp2)


def _pad_idx(v):
    v2 = v.reshape(NBLK, EB)
    v2 = jnp.pad(v2, ((0, 0), (0, EBP - EB)))
    return v2.reshape(EP // IB, IB // 128, 128)


def kernel(x, e, Wn1, bn1, Wn2, bn2, We1, be1, We2, be2, W_gnn, b_gnn,
           ln_ng, ln_nb, ln_eg, ln_eb, Wp1, bp1, Wp2, bp2, edge_index):
    src3d = _pad_idx(edge_index[0].astype(jnp.int32))
    dst3d = _pad_idx(edge_index[1].astype(jnp.int32))
    zrows = jnp.zeros((ZR, W), jnp.float32)

    row = lambda v: v.reshape(1, -1)

    h = _node_enc(x, Wn1, row(bn1), Wn2, row(bn2))
    ee = _edge_enc(e, We1, row(be1), We2, row(be2))

    num_layers = W_gnn.shape[0]
    for l in range(num_layers):
        wa, wb, wc, wd, we = (W_gnn[l, k] for k in range(5))
        ba, bb, bc, bd, be_ = (row(b_gnn[l, k]) for k in range(5))
        ax, t1, t2 = _node_mats(h, wa, wb, wd, we, ba, bb, bd, be_)
        g1_p, g2_p = _sc_gather2(t1, t2, src3d, dst3d)
        ee, ms_p = _edge_update(g1_p, g2_p, ee, wc, bc,
                                row(ln_eg[l]), row(ln_eb[l]))
        acc = _sc_scatter(ms_p, dst3d, zrows)
        h = _node_update(h, ax, acc, row(ln_ng[l]), row(ln_nb[l]))

    tp = _pred_node(h, Wp1[0:H], Wp1[H:2 * H])
    gp1_p, gp2_p = _sc_gather2(tp, tp, src3d, dst3d)
    scores = _pred_final(gp1_p, gp2_p, ee, Wp1[2 * H:3 * H], row(bp1), Wp2,
                         row(bp2).reshape(1, 1))
    return scores


# speedup vs baseline: 5.0519x; 1.0040x over previous
import jax
import jax.numpy as jnp
from jax import lax
from jax.experimental import pallas as pl
from jax.experimental.pallas import tpu as pltpu
from jax.experimental.pallas import tpu_sc as plsc

N = 10000
E = 320000
H = 64
W = 2 * H
EB = 2000
EBP = 2048
NBLK = E // EB
EP = EBP * NBLK
NW = 32
IB = 1024
NCH = EP // (NW * IB)
CHG = 128
NPT = 624
ZR = 208

_MESH = dict(core_axis_name="c", subcore_axis_name="s", num_cores=2,
             num_subcores=16)


NST = 624


def _sc_gather2_body(t1_hbm, t2_hbm, src_hbm, dst_hbm, g1_hbm, g2_hbm,
                     idx0, idx1, buf0, buf1, tabsh, sem_i, sem_g, sem_w):
    c = lax.axis_index("c")
    s = lax.axis_index("s")
    wid = s * 2 + c
    bufs = (buf0, buf1)
    idxs = (idx0, idx1)

    rpb = IB // CHG
    t0 = s * NST
    for tab, idx_hbm, out_hbm in ((t1_hbm, src_hbm, g1_hbm),
                                  (t2_hbm, dst_hbm, g2_hbm)):
        pltpu.sync_copy(tab.at[pl.ds(t0, NST)], tabsh.at[pl.ds(t0, NST)])

        @pl.when(s == 15)
        def _():
            pltpu.sync_copy(tab.at[pl.ds(16 * NST, N - 16 * NST)],
                            tabsh.at[pl.ds(16 * NST, N - 16 * NST)])
        plsc.subcore_barrier()

        icps = [None] * NCH
        wcps = [None, None]
        icps[0] = pltpu.async_copy(idx_hbm.at[wid * NCH], idxs[0], sem_i)
        for r in range(rpb * NCH):
            b = r // rpb
            if r % rpb == 0:
                if b + 1 < NCH:
                    icps[b + 1] = pltpu.async_copy(
                        idx_hbm.at[wid * NCH + b + 1], idxs[(b + 1) % 2],
                        sem_i)
                icps[b].wait()
            if wcps[r % 2] is not None:
                wcps[r % 2].wait()
            gcps = [pltpu.async_copy(
                tabsh.at[idxs[b % 2].at[(r % rpb) * (CHG // 128) + j]],
                bufs[r % 2].at[pl.ds(j * 128, 128)], sem_g)
                for j in range(CHG // 128)]
            for cp in gcps:
                cp.wait()
            e0 = (wid * NCH + b) * IB + (r % rpb) * CHG
            wcps[r % 2] = pltpu.async_copy(
                bufs[r % 2], out_hbm.at[pl.ds(e0, CHG)], sem_w)
        for cp in wcps:
            if cp is not None:
                cp.wait()
        plsc.subcore_barrier()


def _sc_gather2(t1, t2, src3d, dst3d):
    sds = jax.ShapeDtypeStruct((EP, W), jnp.float32)
    f = pl.kernel(
        _sc_gather2_body,
        out_type=(sds, sds),
        mesh=plsc.VectorSubcoreMesh(**_MESH),
        scratch_types=[
            pltpu.VMEM((IB // 128, 128), jnp.int32),
            pltpu.VMEM((IB // 128, 128), jnp.int32),
            pltpu.VMEM((CHG, W), jnp.float32),
            pltpu.VMEM((CHG, W), jnp.float32),
            pltpu.VMEM_SHARED((N, W), jnp.float32),
            pltpu.SemaphoreType.DMA,
            pltpu.SemaphoreType.DMA,
            pltpu.SemaphoreType.DMA,
        ],
    )
    return f(t1, t2, src3d, dst3d)


CHS = 128
NRS = NCH * (IB // CHS)


def _sc_scatter_body(ms_hbm, idx_hbm, z_hbm, acc_out,
                     idx0, idx1, sb0, sb1, acc_sh, sem_i, sem_l, sem_s):
    c = lax.axis_index("c")
    s = lax.axis_index("s")
    wid = s * 2 + c

    r0 = s * NPT
    for k in range(3):
        pltpu.sync_copy(z_hbm, acc_sh.at[pl.ds(r0 + k * ZR, ZR)])

    @pl.when(s == 15)
    def _():
        pltpu.sync_copy(z_hbm.at[pl.ds(0, 16)], acc_sh.at[pl.ds(16 * NPT, 16)])
    plsc.subcore_barrier()

    idxs = (idx0, idx1)
    sbufs = (sb0, sb1)
    rpb = IB // CHS
    icps = [None] * NCH
    lcps = [None, None]
    scps = [None, None]
    icps[0] = pltpu.async_copy(idx_hbm.at[wid * NCH], idxs[0], sem_i)
    lcps[0] = pltpu.async_copy(
        ms_hbm.at[pl.ds(wid * NCH * IB, CHS)], sbufs[0], sem_l)
    for r in range(NRS):
        b = r // rpb
        if r % rpb == 0:
            if b + 1 < NCH:
                icps[b + 1] = pltpu.async_copy(
                    idx_hbm.at[wid * NCH + b + 1], idxs[(b + 1) % 2],
                    sem_i)
            icps[b].wait()
        lcps[r % 2].wait()
        if scps[(r + 1) % 2] is not None:
            scps[(r + 1) % 2].wait()
            scps[(r + 1) % 2] = None
        if r + 1 < NRS:
            e0 = wid * NCH * IB + (r + 1) * CHS
            lcps[(r + 1) % 2] = pltpu.async_copy(
                ms_hbm.at[pl.ds(e0, CHS)], sbufs[(r + 1) % 2], sem_l)
        scps[r % 2] = pltpu.async_copy(
            sbufs[r % 2], acc_sh.at[idxs[b % 2].at[r % rpb]],
            sem_s, add=True)
    for cp in scps:
        if cp is not None:
            cp.wait()

    plsc.subcore_barrier()
    pltpu.sync_copy(acc_sh.at[pl.ds(r0, NPT)], acc_out.at[c, pl.ds(r0, NPT)])

    @pl.when(s == 15)
    def _():
        pltpu.sync_copy(acc_sh.at[pl.ds(16 * NPT, 16)],
                        acc_out.at[c, pl.ds(16 * NPT, 16)])


def _sc_scatter(ms_p, idx3d, zrows):
    f = pl.kernel(
        _sc_scatter_body,
        out_type=jax.ShapeDtypeStruct((2, N, W), jnp.float32),
        mesh=plsc.VectorSubcoreMesh(**_MESH),
        scratch_types=[
            pltpu.VMEM((IB // 128, 128), jnp.int32),
            pltpu.VMEM((IB // 128, 128), jnp.int32),
            pltpu.VMEM((CHS, W), jnp.float32),
            pltpu.VMEM((CHS, W), jnp.float32),
            pltpu.VMEM_SHARED((N, W), jnp.float32),
            pltpu.SemaphoreType.DMA,
            pltpu.SemaphoreType.DMA,
            pltpu.SemaphoreType.DMA,
        ],
    )
    return f(ms_p, idx3d, zrows)


def _ln(v, g, b):
    mu = jnp.mean(v, axis=-1, keepdims=True)
    var = jnp.mean((v - mu) * (v - mu), axis=-1, keepdims=True)
    return g * (v - mu) * lax.rsqrt(var + 1e-5) + b


def _tc_enc_body(x_ref, w1_ref, b1_ref, w2_ref, b2_ref, o_ref):
    hh = jnp.maximum(
        jnp.dot(x_ref[:], w1_ref[:], preferred_element_type=jnp.float32)
        + b1_ref[:], 0.0)
    o_ref[:] = jnp.dot(hh, w2_ref[:], preferred_element_type=jnp.float32) + b2_ref[:]


def _node_enc(x, w1, b1, w2, b2):
    return pl.pallas_call(
        _tc_enc_body,
        out_shape=jax.ShapeDtypeStruct((N, H), jnp.float32),
    )(x, w1, b1, w2, b2)


def _edge_enc(e, w1, b1, w2, b2):
    d_edge = e.shape[1]
    return pl.pallas_call(
        _tc_enc_body,
        grid=(NBLK,),
        in_specs=[
            pl.BlockSpec((EB, d_edge), lambda i: (i, 0)),
            pl.BlockSpec((d_edge, H), lambda i: (0, 0)),
            pl.BlockSpec((1, H), lambda i: (0, 0)),
            pl.BlockSpec((H, H), lambda i: (0, 0)),
            pl.BlockSpec((1, H), lambda i: (0, 0)),
        ],
        out_specs=pl.BlockSpec((EB, H), lambda i: (i, 0)),
        out_shape=jax.ShapeDtypeStruct((E, H), jnp.float32),
    )(e, w1, b1, w2, b2)


def _tc_node_mats_body(h_ref, wa, wb, wd, we, ba, bb, bd, be,
                       ax_o, t1_o, t2_o):
    h = h_ref[:]
    ax_o[:] = jnp.dot(h, wa[:], preferred_element_type=jnp.float32) + ba[:]
    bx = jnp.dot(h, wb[:], preferred_element_type=jnp.float32) + bb[:]
    dx = jnp.dot(h, wd[:], preferred_element_type=jnp.float32) + bd[:]
    ex = jnp.dot(h, we[:], preferred_element_type=jnp.float32) + be[:]
    t1_o[:] = jnp.concatenate([dx, bx], axis=1)
    t2_o[:] = jnp.concatenate([ex, jnp.zeros((N, H), jnp.float32)], axis=1)


def _node_mats(h, wa, wb, wd, we, ba, bb, bd, be):
    return pl.pallas_call(
        _tc_node_mats_body,
        out_shape=(jax.ShapeDtypeStruct((N, H), jnp.float32),
                   jax.ShapeDtypeStruct((N, W), jnp.float32),
                   jax.ShapeDtypeStruct((N, W), jnp.float32)),
    )(h, wa, wb, wd, we, ba, bb, bd, be)


def _tc_edge_update_body(g1_ref, g2_ref, ee_ref, c_ref, bc_ref, eg_ref,
                         eb_ref, eeo_ref, ms_ref):
    ee = ee_ref[:]
    ce = jnp.dot(ee, c_ref[:], preferred_element_type=jnp.float32) + bc_ref[:]
    g1 = g1_ref[0:EB, :]
    epre = g1[:, 0:H] + g2_ref[0:EB, 0:H] + ce
    eeo_ref[:] = ee + jnp.maximum(_ln(epre, eg_ref[:], eb_ref[:]), 0.0)
    sig = jax.nn.sigmoid(epre)
    msg = sig * g1[:, H:W]
    ms_ref[0:EB, :] = jnp.concatenate([msg, sig], axis=1)
    ms_ref[EB:EBP, :] = jnp.zeros((EBP - EB, W), jnp.float32)


def _edge_update(g1_p, g2_p, ee, wc, bc, eg, eb):
    return pl.pallas_call(
        _tc_edge_update_body,
        grid=(NBLK,),
        in_specs=[
            pl.BlockSpec((EBP, W), lambda i: (i, 0)),
            pl.BlockSpec((EBP, W), lambda i: (i, 0)),
            pl.BlockSpec((EB, H), lambda i: (i, 0)),
            pl.BlockSpec((H, H), lambda i: (0, 0)),
            pl.BlockSpec((1, H), lambda i: (0, 0)),
            pl.BlockSpec((1, H), lambda i: (0, 0)),
            pl.BlockSpec((1, H), lambda i: (0, 0)),
        ],
        out_specs=(pl.BlockSpec((EB, H), lambda i: (i, 0)),
                   pl.BlockSpec((EBP, W), lambda i: (i, 0))),
        out_shape=(jax.ShapeDtypeStruct((E, H), jnp.float32),
                   jax.ShapeDtypeStruct((EP, W), jnp.float32)),
    )(g1_p, g2_p, ee, wc, bc, eg, eb)


def _new_h(h_ref, ax_ref, acc_ref, g_ref, b_ref):
    acc = acc_ref[0] + acc_ref[1]
    agg = acc[:, 0:H] / (acc[:, H:W] + 1e-6)
    return h_ref[:] + jnp.maximum(
        _ln(ax_ref[:] + agg, g_ref[:], b_ref[:]), 0.0)


def _tc_update_mats_body(h_ref, ax_ref, acc_ref, g_ref, b_ref,
                         wa, wb, wd, we, ba, bb, bd, be,
                         h_o, ax_o, t1_o, t2_o):
    h = _new_h(h_ref, ax_ref, acc_ref, g_ref, b_ref)
    h_o[:] = h
    ax_o[:] = jnp.dot(h, wa[:], preferred_element_type=jnp.float32) + ba[:]
    bx = jnp.dot(h, wb[:], preferred_element_type=jnp.float32) + bb[:]
    dx = jnp.dot(h, wd[:], preferred_element_type=jnp.float32) + bd[:]
    ex = jnp.dot(h, we[:], preferred_element_type=jnp.float32) + be[:]
    t1_o[:] = jnp.concatenate([dx, bx], axis=1)
    t2_o[:] = jnp.concatenate([ex, jnp.zeros((N, H), jnp.float32)], axis=1)


def _update_mats(h, ax, acc, g, b, wa, wb, wd, we, ba, bb, bd, be):
    return pl.pallas_call(
        _tc_update_mats_body,
        out_shape=(jax.ShapeDtypeStruct((N, H), jnp.float32),
                   jax.ShapeDtypeStruct((N, H), jnp.float32),
                   jax.ShapeDtypeStruct((N, W), jnp.float32),
                   jax.ShapeDtypeStruct((N, W), jnp.float32)),
    )(h, ax, acc, g, b, wa, wb, wd, we, ba, bb, bd, be)


def _tc_update_pred_body(h_ref, ax_ref, acc_ref, g_ref, b_ref,
                         wa_ref, wb_ref, tp_o):
    h = _new_h(h_ref, ax_ref, acc_ref, g_ref, b_ref)
    p1 = jnp.dot(h, wa_ref[:], preferred_element_type=jnp.float32)
    p2 = jnp.dot(h, wb_ref[:], preferred_element_type=jnp.float32)
    tp_o[:] = jnp.concatenate([p1, p2], axis=1)


def _update_pred(h, ax, acc, g, b, w1a, w1b):
    return pl.pallas_call(
        _tc_update_pred_body,
        out_shape=jax.ShapeDtypeStruct((N, W), jnp.float32),
    )(h, ax, acc, g, b, w1a, w1b)


def _tc_pred_final_body(gp1_ref, gp2_ref, ee_ref, w1c_ref, bp1_ref, wp2_ref,
                        bp2_ref, o_ref):
    z = jnp.maximum(
        gp1_ref[0:EB, 0:H] + gp2_ref[0:EB, H:W]
        + jnp.dot(ee_ref[:], w1c_ref[:], preferred_element_type=jnp.float32)
        + bp1_ref[:], 0.0)
    o_ref[:] = jnp.dot(z, wp2_ref[:], preferred_element_type=jnp.float32) + bp2_ref[:]


def _pred_final(gp1_p, gp2_p, ee, w1c, bp1, wp2, bp2):
    return pl.pallas_call(
        _tc_pred_final_body,
        grid=(NBLK,),
        in_specs=[
            pl.BlockSpec((EBP, W), lambda i: (i, 0)),
            pl.BlockSpec((EBP, W), lambda i: (i, 0)),
            pl.BlockSpec((EB, H), lambda i: (i, 0)),
            pl.BlockSpec((H, H), lambda i: (0, 0)),
            pl.BlockSpec((1, H), lambda i: (0, 0)),
            pl.BlockSpec((H, 1), lambda i: (0, 0)),
            pl.BlockSpec((1, 1), lambda i: (0, 0)),
        ],
        out_specs=pl.BlockSpec((EB, 1), lambda i: (i, 0)),
        out_shape=jax.ShapeDtypeStruct((E, 1), jnp.float32),
    )(gp1_p, gp2_p, ee, w1c, bp1, wp2, bp2)


def _pad_idx(v):
    v2 = v.reshape(NBLK, EB)
    v2 = jnp.pad(v2, ((0, 0), (0, EBP - EB)))
    return v2.reshape(EP // IB, IB // 128, 128)


def kernel(x, e, Wn1, bn1, Wn2, bn2, We1, be1, We2, be2, W_gnn, b_gnn,
           ln_ng, ln_nb, ln_eg, ln_eb, Wp1, bp1, Wp2, bp2, edge_index):
    src3d = _pad_idx(edge_index[0].astype(jnp.int32))
    dst3d = _pad_idx(edge_index[1].astype(jnp.int32))
    zrows = jnp.zeros((ZR, W), jnp.float32)

    row = lambda v: v.reshape(1, -1)

    h = _node_enc(x, Wn1, row(bn1), Wn2, row(bn2))
    ee = _edge_enc(e, We1, row(be1), We2, row(be2))

    num_layers = W_gnn.shape[0]

    def wts(l):
        wa, wb, wc, wd, we = (W_gnn[l, k] for k in range(5))
        ba, bb, bc, bd, be_ = (row(b_gnn[l, k]) for k in range(5))
        return wa, wb, wc, wd, we, ba, bb, bc, bd, be_

    wa, wb, wc, wd, we, ba, bb, bc, bd, be_ = wts(0)
    ax, t1, t2 = _node_mats(h, wa, wb, wd, we, ba, bb, bd, be_)
    for l in range(num_layers):
        g1_p, g2_p = _sc_gather2(t1, t2, src3d, dst3d)
        ee, ms_p = _edge_update(g1_p, g2_p, ee, wc, bc,
                                row(ln_eg[l]), row(ln_eb[l]))
        acc = _sc_scatter(ms_p, dst3d, zrows)
        g, b = row(ln_ng[l]), row(ln_nb[l])
        if l + 1 < num_layers:
            wa, wb, wc, wd, we, ba, bb, bc, bd, be_ = wts(l + 1)
            h, ax, t1, t2 = _update_mats(h, ax, acc, g, b,
                                         wa, wb, wd, we, ba, bb, bd, be_)
        else:
            tp = _update_pred(h, ax, acc, g, b, Wp1[0:H], Wp1[H:2 * H])

    gp1_p, gp2_p = _sc_gather2(tp, tp, src3d, dst3d)
    scores = _pred_final(gp1_p, gp2_p, ee, Wp1[2 * H:3 * H], row(bp1), Wp2,
                         row(bp2).reshape(1, 1))
    return scores


# speedup vs baseline: 5.2882x; 1.0468x over previous
import jax
import jax.numpy as jnp
from jax import lax
from jax.experimental import pallas as pl
from jax.experimental.pallas import tpu as pltpu
from jax.experimental.pallas import tpu_sc as plsc

N = 10000
E = 320000
H = 64
W = 2 * H
EB = 2000
EBP = 2048
NBLK = E // EB
EP = EBP * NBLK
NW = 32
IB = 1024
NCH = EP // (NW * IB)
CHG = 128
NPT = 624
ZR = 208

_MESH = dict(core_axis_name="c", subcore_axis_name="s", num_cores=2,
             num_subcores=16)


NST = 624


def _make_gather2_body(nch):
  def _sc_gather2_body(t1_hbm, t2_hbm, src_hbm, dst_hbm, g1_hbm, g2_hbm,
                     idx0, idx1, buf0, buf1, tabsh, sem_i, sem_g, sem_w):
    NCH = nch
    c = lax.axis_index("c")
    s = lax.axis_index("s")
    wid = s * 2 + c
    bufs = (buf0, buf1)
    idxs = (idx0, idx1)

    rpb = IB // CHG
    t0 = s * NST
    for tab, idx_hbm, out_hbm in ((t1_hbm, src_hbm, g1_hbm),
                                  (t2_hbm, dst_hbm, g2_hbm)):
        pltpu.sync_copy(tab.at[pl.ds(t0, NST)], tabsh.at[pl.ds(t0, NST)])

        @pl.when(s == 15)
        def _():
            pltpu.sync_copy(tab.at[pl.ds(16 * NST, N - 16 * NST)],
                            tabsh.at[pl.ds(16 * NST, N - 16 * NST)])
        plsc.subcore_barrier()

        icps = [None] * NCH
        wcps = [None, None]
        icps[0] = pltpu.async_copy(idx_hbm.at[wid * NCH], idxs[0], sem_i)
        for r in range(rpb * NCH):
            b = r // rpb
            if r % rpb == 0:
                if b + 1 < NCH:
                    icps[b + 1] = pltpu.async_copy(
                        idx_hbm.at[wid * NCH + b + 1], idxs[(b + 1) % 2],
                        sem_i)
                icps[b].wait()
            if wcps[r % 2] is not None:
                wcps[r % 2].wait()
            gcps = [pltpu.async_copy(
                tabsh.at[idxs[b % 2].at[(r % rpb) * (CHG // 128) + j]],
                bufs[r % 2].at[pl.ds(j * 128, 128)], sem_g)
                for j in range(CHG // 128)]
            for cp in gcps:
                cp.wait()
            e0 = (wid * NCH + b) * IB + (r % rpb) * CHG
            wcps[r % 2] = pltpu.async_copy(
                bufs[r % 2], out_hbm.at[pl.ds(e0, CHG)], sem_w)
        for cp in wcps:
            if cp is not None:
                cp.wait()
        plsc.subcore_barrier()
  return _sc_gather2_body


def _sc_gather2(t1, t2, src3d, dst3d):
    nch = src3d.shape[0] // NW
    sds = jax.ShapeDtypeStruct((src3d.shape[0] * IB, W), jnp.float32)
    f = pl.kernel(
        _make_gather2_body(nch),
        out_type=(sds, sds),
        mesh=plsc.VectorSubcoreMesh(**_MESH),
        scratch_types=[
            pltpu.VMEM((IB // 128, 128), jnp.int32),
            pltpu.VMEM((IB // 128, 128), jnp.int32),
            pltpu.VMEM((CHG, W), jnp.float32),
            pltpu.VMEM((CHG, W), jnp.float32),
            pltpu.VMEM_SHARED((N, W), jnp.float32),
            pltpu.SemaphoreType.DMA,
            pltpu.SemaphoreType.DMA,
            pltpu.SemaphoreType.DMA,
        ],
    )
    return f(t1, t2, src3d, dst3d)


CHS = 128
NRS = NCH * (IB // CHS)


def _make_scatter_body(nch):
  def _sc_scatter_body(ms_hbm, idx_hbm, z_hbm, acc_out,
                     idx0, idx1, sb0, sb1, acc_sh, sem_i, sem_l, sem_s):
    NCH = nch
    NRS = NCH * (IB // CHS)
    c = lax.axis_index("c")
    s = lax.axis_index("s")
    wid = s * 2 + c

    r0 = s * NPT
    for k in range(3):
        pltpu.sync_copy(z_hbm, acc_sh.at[pl.ds(r0 + k * ZR, ZR)])

    @pl.when(s == 15)
    def _():
        pltpu.sync_copy(z_hbm.at[pl.ds(0, 16)], acc_sh.at[pl.ds(16 * NPT, 16)])
    plsc.subcore_barrier()

    idxs = (idx0, idx1)
    sbufs = (sb0, sb1)
    rpb = IB // CHS
    icps = [None] * NCH
    lcps = [None, None]
    scps = [None, None]
    icps[0] = pltpu.async_copy(idx_hbm.at[wid * NCH], idxs[0], sem_i)
    lcps[0] = pltpu.async_copy(
        ms_hbm.at[pl.ds(wid * NCH * IB, CHS)], sbufs[0], sem_l)
    for r in range(NRS):
        b = r // rpb
        if r % rpb == 0:
            if b + 1 < NCH:
                icps[b + 1] = pltpu.async_copy(
                    idx_hbm.at[wid * NCH + b + 1], idxs[(b + 1) % 2],
                    sem_i)
            icps[b].wait()
        lcps[r % 2].wait()
        if scps[(r + 1) % 2] is not None:
            scps[(r + 1) % 2].wait()
            scps[(r + 1) % 2] = None
        if r + 1 < NRS:
            e0 = wid * NCH * IB + (r + 1) * CHS
            lcps[(r + 1) % 2] = pltpu.async_copy(
                ms_hbm.at[pl.ds(e0, CHS)], sbufs[(r + 1) % 2], sem_l)
        scps[r % 2] = pltpu.async_copy(
            sbufs[r % 2], acc_sh.at[idxs[b % 2].at[r % rpb]],
            sem_s, add=True)
    for cp in scps:
        if cp is not None:
            cp.wait()

    plsc.subcore_barrier()
    pltpu.sync_copy(acc_sh.at[pl.ds(r0, NPT)], acc_out.at[c, pl.ds(r0, NPT)])

    @pl.when(s == 15)
    def _():
        pltpu.sync_copy(acc_sh.at[pl.ds(16 * NPT, 16)],
                        acc_out.at[c, pl.ds(16 * NPT, 16)])
  return _sc_scatter_body


def _sc_scatter(ms_p, idx3d, zrows):
    f = pl.kernel(
        _make_scatter_body(ms_p.shape[0] // (NW * IB)),
        out_type=jax.ShapeDtypeStruct((2, N, W), jnp.float32),
        mesh=plsc.VectorSubcoreMesh(**_MESH),
        scratch_types=[
            pltpu.VMEM((IB // 128, 128), jnp.int32),
            pltpu.VMEM((IB // 128, 128), jnp.int32),
            pltpu.VMEM((CHS, W), jnp.float32),
            pltpu.VMEM((CHS, W), jnp.float32),
            pltpu.VMEM_SHARED((N, W), jnp.float32),
            pltpu.SemaphoreType.DMA,
            pltpu.SemaphoreType.DMA,
            pltpu.SemaphoreType.DMA,
        ],
    )
    return f(ms_p, idx3d, zrows)


def _ln(v, g, b):
    mu = jnp.mean(v, axis=-1, keepdims=True)
    var = jnp.mean((v - mu) * (v - mu), axis=-1, keepdims=True)
    return g * (v - mu) * lax.rsqrt(var + 1e-5) + b


def _tc_enc_body(x_ref, w1_ref, b1_ref, w2_ref, b2_ref, o_ref):
    hh = jnp.maximum(
        jnp.dot(x_ref[:], w1_ref[:], preferred_element_type=jnp.float32)
        + b1_ref[:], 0.0)
    o_ref[:] = jnp.dot(hh, w2_ref[:], preferred_element_type=jnp.float32) + b2_ref[:]


def _node_enc(x, w1, b1, w2, b2):
    return pl.pallas_call(
        _tc_enc_body,
        out_shape=jax.ShapeDtypeStruct((N, H), jnp.float32),
    )(x, w1, b1, w2, b2)


def _edge_enc(e, w1, b1, w2, b2, base):
    d_edge = e.shape[1]
    return pl.pallas_call(
        _tc_enc_body,
        grid=(NBLK // 2,),
        in_specs=[
            pl.BlockSpec((EB, d_edge), lambda i: (i + base, 0)),
            pl.BlockSpec((d_edge, H), lambda i: (0, 0)),
            pl.BlockSpec((1, H), lambda i: (0, 0)),
            pl.BlockSpec((H, H), lambda i: (0, 0)),
            pl.BlockSpec((1, H), lambda i: (0, 0)),
        ],
        out_specs=pl.BlockSpec((EB, H), lambda i: (i, 0)),
        out_shape=jax.ShapeDtypeStruct((E // 2, H), jnp.float32),
    )(e, w1, b1, w2, b2)


def _tc_node_mats_body(h_ref, wa, wb, wd, we, ba, bb, bd, be,
                       ax_o, t1_o, t2_o):
    h = h_ref[:]
    ax_o[:] = jnp.dot(h, wa[:], preferred_element_type=jnp.float32) + ba[:]
    bx = jnp.dot(h, wb[:], preferred_element_type=jnp.float32) + bb[:]
    dx = jnp.dot(h, wd[:], preferred_element_type=jnp.float32) + bd[:]
    ex = jnp.dot(h, we[:], preferred_element_type=jnp.float32) + be[:]
    t1_o[:] = jnp.concatenate([dx, bx], axis=1)
    t2_o[:] = jnp.concatenate([ex, jnp.zeros((N, H), jnp.float32)], axis=1)


def _node_mats(h, wa, wb, wd, we, ba, bb, bd, be):
    return pl.pallas_call(
        _tc_node_mats_body,
        out_shape=(jax.ShapeDtypeStruct((N, H), jnp.float32),
                   jax.ShapeDtypeStruct((N, W), jnp.float32),
                   jax.ShapeDtypeStruct((N, W), jnp.float32)),
    )(h, wa, wb, wd, we, ba, bb, bd, be)


def _tc_edge_update_body(g1_ref, g2_ref, ee_ref, c_ref, bc_ref, eg_ref,
                         eb_ref, eeo_ref, ms_ref):
    ee = ee_ref[:]
    ce = jnp.dot(ee, c_ref[:], preferred_element_type=jnp.float32) + bc_ref[:]
    g1 = g1_ref[0:EB, :]
    epre = g1[:, 0:H] + g2_ref[0:EB, 0:H] + ce
    eeo_ref[:] = ee + jnp.maximum(_ln(epre, eg_ref[:], eb_ref[:]), 0.0)
    sig = jax.nn.sigmoid(epre)
    msg = sig * g1[:, H:W]
    ms_ref[0:EB, :] = jnp.concatenate([msg, sig], axis=1)
    ms_ref[EB:EBP, :] = jnp.zeros((EBP - EB, W), jnp.float32)


def _edge_update(g1_p, g2_p, ee, wc, bc, eg, eb):
    return pl.pallas_call(
        _tc_edge_update_body,
        grid=(NBLK // 2,),
        in_specs=[
            pl.BlockSpec((EBP, W), lambda i: (i, 0)),
            pl.BlockSpec((EBP, W), lambda i: (i, 0)),
            pl.BlockSpec((EB, H), lambda i: (i, 0)),
            pl.BlockSpec((H, H), lambda i: (0, 0)),
            pl.BlockSpec((1, H), lambda i: (0, 0)),
            pl.BlockSpec((1, H), lambda i: (0, 0)),
            pl.BlockSpec((1, H), lambda i: (0, 0)),
        ],
        out_specs=(pl.BlockSpec((EB, H), lambda i: (i, 0)),
                   pl.BlockSpec((EBP, W), lambda i: (i, 0))),
        out_shape=(jax.ShapeDtypeStruct((E // 2, H), jnp.float32),
                   jax.ShapeDtypeStruct((EP // 2, W), jnp.float32)),
    )(g1_p, g2_p, ee, wc, bc, eg, eb)


def _new_h(h_ref, ax_ref, acca_ref, accb_ref, g_ref, b_ref):
    acc = (acca_ref[0] + acca_ref[1]) + (accb_ref[0] + accb_ref[1])
    agg = acc[:, 0:H] / (acc[:, H:W] + 1e-6)
    return h_ref[:] + jnp.maximum(
        _ln(ax_ref[:] + agg, g_ref[:], b_ref[:]), 0.0)


def _tc_update_mats_body(h_ref, ax_ref, acca_ref, accb_ref, g_ref, b_ref,
                         wa, wb, wd, we, ba, bb, bd, be,
                         h_o, ax_o, t1_o, t2_o):
    h = _new_h(h_ref, ax_ref, acca_ref, accb_ref, g_ref, b_ref)
    h_o[:] = h
    ax_o[:] = jnp.dot(h, wa[:], preferred_element_type=jnp.float32) + ba[:]
    bx = jnp.dot(h, wb[:], preferred_element_type=jnp.float32) + bb[:]
    dx = jnp.dot(h, wd[:], preferred_element_type=jnp.float32) + bd[:]
    ex = jnp.dot(h, we[:], preferred_element_type=jnp.float32) + be[:]
    t1_o[:] = jnp.concatenate([dx, bx], axis=1)
    t2_o[:] = jnp.concatenate([ex, jnp.zeros((N, H), jnp.float32)], axis=1)


def _update_mats(h, ax, acca, accb, g, b, wa, wb, wd, we, ba, bb, bd, be):
    return pl.pallas_call(
        _tc_update_mats_body,
        out_shape=(jax.ShapeDtypeStruct((N, H), jnp.float32),
                   jax.ShapeDtypeStruct((N, H), jnp.float32),
                   jax.ShapeDtypeStruct((N, W), jnp.float32),
                   jax.ShapeDtypeStruct((N, W), jnp.float32)),
    )(h, ax, acca, accb, g, b, wa, wb, wd, we, ba, bb, bd, be)


def _tc_update_pred_body(h_ref, ax_ref, acca_ref, accb_ref, g_ref, b_ref,
                         wa_ref, wb_ref, tp_o):
    h = _new_h(h_ref, ax_ref, acca_ref, accb_ref, g_ref, b_ref)
    p1 = jnp.dot(h, wa_ref[:], preferred_element_type=jnp.float32)
    p2 = jnp.dot(h, wb_ref[:], preferred_element_type=jnp.float32)
    tp_o[:] = jnp.concatenate([p1, p2], axis=1)


def _update_pred(h, ax, acca, accb, g, b, w1a, w1b):
    return pl.pallas_call(
        _tc_update_pred_body,
        out_shape=jax.ShapeDtypeStruct((N, W), jnp.float32),
    )(h, ax, acca, accb, g, b, w1a, w1b)


def _tc_pred_final_body(gp1_ref, gp2_ref, ee_ref, w1c_ref, bp1_ref, wp2_ref,
                        bp2_ref, o_ref):
    z = jnp.maximum(
        gp1_ref[0:EB, 0:H] + gp2_ref[0:EB, H:W]
        + jnp.dot(ee_ref[:], w1c_ref[:], preferred_element_type=jnp.float32)
        + bp1_ref[:], 0.0)
    o_ref[:] = jnp.dot(z, wp2_ref[:], preferred_element_type=jnp.float32) + bp2_ref[:]


def _pred_final(gp1_p, gp2_p, ee, w1c, bp1, wp2, bp2, base):
    return pl.pallas_call(
        _tc_pred_final_body,
        grid=(NBLK // 2,),
        in_specs=[
            pl.BlockSpec((EBP, W), lambda i: (i + base, 0)),
            pl.BlockSpec((EBP, W), lambda i: (i + base, 0)),
            pl.BlockSpec((EB, H), lambda i: (i, 0)),
            pl.BlockSpec((H, H), lambda i: (0, 0)),
            pl.BlockSpec((1, H), lambda i: (0, 0)),
            pl.BlockSpec((H, 1), lambda i: (0, 0)),
            pl.BlockSpec((1, 1), lambda i: (0, 0)),
        ],
        out_specs=pl.BlockSpec((EB, 1), lambda i: (i, 0)),
        out_shape=jax.ShapeDtypeStruct((E // 2, 1), jnp.float32),
    )(gp1_p, gp2_p, ee, w1c, bp1, wp2, bp2)


def _pad_idx(v):
    v2 = v.reshape(NBLK, EB)
    v2 = jnp.pad(v2, ((0, 0), (0, EBP - EB)))
    return v2.reshape(EP // IB, IB // 128, 128)


def kernel(x, e, Wn1, bn1, Wn2, bn2, We1, be1, We2, be2, W_gnn, b_gnn,
           ln_ng, ln_nb, ln_eg, ln_eb, Wp1, bp1, Wp2, bp2, edge_index):
    src3d = _pad_idx(edge_index[0].astype(jnp.int32))
    dst3d = _pad_idx(edge_index[1].astype(jnp.int32))
    zrows = jnp.zeros((ZR, W), jnp.float32)

    row = lambda v: v.reshape(1, -1)

    nib = EP // IB
    srcA, srcB = src3d[:nib // 2], src3d[nib // 2:]
    dstA, dstB = dst3d[:nib // 2], dst3d[nib // 2:]

    h = _node_enc(x, Wn1, row(bn1), Wn2, row(bn2))
    ee_a = _edge_enc(e, We1, row(be1), We2, row(be2), 0)
    ee_b = _edge_enc(e, We1, row(be1), We2, row(be2), NBLK // 2)

    num_layers = W_gnn.shape[0]

    def wts(l):
        wa, wb, wc, wd, we = (W_gnn[l, k] for k in range(5))
        ba, bb, bc, bd, be_ = (row(b_gnn[l, k]) for k in range(5))
        return wa, wb, wc, wd, we, ba, bb, bc, bd, be_

    wa, wb, wc, wd, we, ba, bb, bc, bd, be_ = wts(0)
    ax, t1, t2 = _node_mats(h, wa, wb, wd, we, ba, bb, bd, be_)
    for l in range(num_layers):
        eg, ebb = row(ln_eg[l]), row(ln_eb[l])
        gA1, gA2 = _sc_gather2(t1, t2, srcA, dstA)
        gB1, gB2 = _sc_gather2(t1, t2, srcB, dstB)
        ee_a, msA = _edge_update(gA1, gA2, ee_a, wc, bc, eg, ebb)
        ee_b, msB = _edge_update(gB1, gB2, ee_b, wc, bc, eg, ebb)
        accA = _sc_scatter(msA, dstA, zrows)
        accB = _sc_scatter(msB, dstB, zrows)
        g, b = row(ln_ng[l]), row(ln_nb[l])
        if l + 1 < num_layers:
            wa, wb, wc, wd, we, ba, bb, bc, bd, be_ = wts(l + 1)
            h, ax, t1, t2 = _update_mats(h, ax, accA, accB, g, b,
                                         wa, wb, wd, we, ba, bb, bd, be_)
        else:
            tp = _update_pred(h, ax, accA, accB, g, b, Wp1[0:H], Wp1[H:2 * H])

    gp1_p, gp2_p = _sc_gather2(tp, tp, src3d, dst3d)
    w1c, bp1r, bp2r = Wp1[2 * H:3 * H], row(bp1), row(bp2).reshape(1, 1)
    sc_a = _pred_final(gp1_p, gp2_p, ee_a, w1c, bp1r, Wp2, bp2r, 0)
    sc_b = _pred_final(gp1_p, gp2_p, ee_b, w1c, bp1r, Wp2, bp2r, NBLK // 2)
    return jnp.concatenate([sc_a, sc_b], axis=0)
